# baseline (speedup 1.0000x reference)
import functools

import jax
import jax.numpy as jnp
from jax import lax
from jax.experimental import pallas as pl
from jax.experimental.pallas import tpu as pltpu
from jax.experimental.pallas import tpu_sc as plsc

KP = 15
KN = 16
NW = 32
RADII = (0.0625, 0.125, 0.25, 0.5)
F32 = jnp.float32


def _leaky(x):
    return jnp.where(x >= 0.0, x, 0.1 * x)


def _pad_rows(x, n):
    if x.shape[0] == n:
        return x
    return jnp.pad(x, ((0, n - x.shape[0]),) + ((0, 0),) * (x.ndim - 1))


def _sc_gather(table, idx_flat):
    _, D = table.shape
    B = idx_flat.shape[0]
    bpw = B // NW
    ch = min(128, bpw)
    while bpw % ch:
        ch -= 8
    nch = bpw // ch
    mesh = plsc.VectorSubcoreMesh(core_axis_name="c", subcore_axis_name="s")

    @functools.partial(
        pl.kernel, mesh=mesh,
        out_type=jax.ShapeDtypeStruct((B, D), F32),
        compiler_params=pltpu.CompilerParams(use_tc_tiling_on_sc=False),
        scratch_types=[pltpu.VMEM((ch,), jnp.int32),
                       pltpu.VMEM((ch, D), F32),
                       pltpu.SemaphoreType.DMA])
    def gk(table_hbm, idx_hbm, out_hbm, idx_v, rows_v, sem):
        wid = lax.axis_index("s") * 2 + lax.axis_index("c")
        base = wid * bpw

        @pl.loop(0, nch)
        def _(i):
            b = base + i * ch
            pltpu.sync_copy(idx_hbm.at[pl.ds(b, ch)], idx_v)
            pltpu.async_copy(table_hbm.at[idx_v], rows_v, sem).wait()
            pltpu.sync_copy(rows_v, out_hbm.at[pl.ds(b, ch)])

    return gk(table, idx_flat)


def _gather_rows(table, idx_flat):
    out = _sc_gather(table, idx_flat)
    return out.reshape(idx_flat.shape[0] // KN, KN * table.shape[1])


def _dot(a, b):
    return jax.lax.dot_general(a, b, (((1,), (0,)), ((), ())),
                               preferred_element_type=F32)


def _infl_cols(g, px, py, pz, kx, ky, kz, inv_r):
    cols = []
    for k in range(KN):
        o = 16 * k
        rx = g[:, o:o + 1] - px
        ry = g[:, o + 1:o + 2] - py
        rz = g[:, o + 2:o + 3] - pz
        dx = rx - kx
        dy = ry - ky
        dz = rz - kz
        d = jnp.sqrt(dx * dx + dy * dy + dz * dz + 1e-12)
        cols.append(jnp.maximum(0.0, 1.0 - d * inv_r))
    return cols


def _tc_level0(g0, t0, kpts, w0, b0, wd, bd, blk):
    n = t0.shape[0]
    inv_r = 1.0 / RADII[0]
    c0, m = w0.shape[1], wd.shape[1]
    kx, ky, kz = (kpts[:, i].reshape(1, KP) for i in range(3))

    def body(g_ref, t_ref, kx_ref, ky_ref, kz_ref, w0_ref, b0_ref,
             wd_ref, bd_ref, infl_ref, x0_ref, y1_ref):
        g = g_ref[...]
        t = t_ref[...]
        px, py, pz = t[:, 0:1], t[:, 1:2], t[:, 2:3]
        cols = _infl_cols(g, px, py, pz, kx_ref[...], ky_ref[...],
                          kz_ref[...], inv_r)
        agg = None
        for k in range(KN):
            a = cols[k] * g[:, 16 * k + 3:16 * k + 4]
            agg = a if agg is None else agg + a
        infl_ref[...] = jnp.concatenate(cols, axis=1)
        x0 = _leaky(_dot(agg, w0_ref[...]) + b0_ref[...])
        x0_ref[...] = x0
        y1_ref[...] = _leaky(_dot(x0, wd_ref[...]) + bd_ref[...])

    wspec = lambda i: (0, 0)
    return pl.pallas_call(
        body,
        grid=(n // blk,),
        in_specs=[pl.BlockSpec((blk, KN * 16), lambda i: (i, 0)),
                  pl.BlockSpec((blk, 16), lambda i: (i, 0)),
                  pl.BlockSpec((1, KP), wspec),
                  pl.BlockSpec((1, KP), wspec),
                  pl.BlockSpec((1, KP), wspec),
                  pl.BlockSpec((KP, c0), wspec),
                  pl.BlockSpec((1, c0), wspec),
                  pl.BlockSpec((c0, m), wspec),
                  pl.BlockSpec((1, m), wspec)],
        out_specs=[pl.BlockSpec((blk, KN * KP), lambda i: (i, 0)),
                   pl.BlockSpec((blk, c0), lambda i: (i, 0)),
                   pl.BlockSpec((blk, m), lambda i: (i, 0))],
        out_shape=[jax.ShapeDtypeStruct((n, KN * KP), F32),
                   jax.ShapeDtypeStruct((n, c0), F32),
                   jax.ShapeDtypeStruct((n, m), F32)],
    )(g0, t0, kx, ky, kz, w0, b0, wd, bd)


def _tc_infl(gpos, posp, kpts, radius, blk):
    n = posp.shape[0]
    inv_r = 1.0 / radius
    kx, ky, kz = (kpts[:, i].reshape(1, KP) for i in range(3))

    def body(g_ref, t_ref, kx_ref, ky_ref, kz_ref, infl_ref):
        g = g_ref[...]
        t = t_ref[...]
        cols = _infl_cols(g, t[:, 0:1], t[:, 1:2], t[:, 2:3],
                          kx_ref[...], ky_ref[...], kz_ref[...], inv_r)
        infl_ref[...] = jnp.concatenate(cols, axis=1)

    wspec = lambda i: (0, 0)
    return pl.pallas_call(
        body,
        grid=(n // blk,),
        in_specs=[pl.BlockSpec((blk, KN * 16), lambda i: (i, 0)),
                  pl.BlockSpec((blk, 16), lambda i: (i, 0)),
                  pl.BlockSpec((1, KP), wspec),
                  pl.BlockSpec((1, KP), wspec),
                  pl.BlockSpec((1, KP), wspec)],
        out_specs=pl.BlockSpec((blk, KN * KP), lambda i: (i, 0)),
        out_shape=jax.ShapeDtypeStruct((n, KN * KP), F32),
    )(gpos, posp, kx, ky, kz)


def _tc_pooldown(gp, wd, bd, blk):
    n = gp.shape[0]
    cp, m = wd.shape

    def body(g_ref, wd_ref, bd_ref, xp_ref, y_ref):
        g = g_ref[...]
        xp = g[:, :cp]
        for k in range(1, KN):
            xp = jnp.maximum(xp, g[:, k * cp:(k + 1) * cp])
        xp_ref[...] = xp
        y_ref[...] = _leaky(_dot(xp, wd_ref[...]) + bd_ref[...])

    wspec = lambda i: (0, 0)
    return pl.pallas_call(
        body,
        grid=(n // blk,),
        in_specs=[pl.BlockSpec((blk, KN * cp), lambda i: (i, 0)),
                  pl.BlockSpec((cp, m), wspec),
                  pl.BlockSpec((1, m), wspec)],
        out_specs=[pl.BlockSpec((blk, cp), lambda i: (i, 0)),
                   pl.BlockSpec((blk, m), lambda i: (i, 0))],
        out_shape=[jax.ShapeDtypeStruct((n, cp), F32),
                   jax.ShapeDtypeStruct((n, m), F32)],
    )(gp, wd, bd)


def _tc_resblock(gy, infl, x_in, wf, bk, wu, bu, wsc, dn, head, blk):
    n = gy.shape[0]
    m = wf.shape[1]
    ci = x_in.shape[1]
    co = wu.shape[1]
    has_sc = wsc is not None
    wdn, bdn = dn if dn is not None else (None, None)
    wh, bh = head if head is not None else (None, None)

    def body(*refs):
        it = iter(refs)
        g_ref, infl_ref, x_ref, wf_ref, bk_ref, wu_ref, bu_ref = (
            next(it) for _ in range(7))
        wsc_ref = next(it) if has_sc else None
        wdn_ref, bdn_ref = (next(it), next(it)) if dn is not None else (None, None)
        wh_ref, bh_ref = (next(it), next(it)) if head is not None else (None, None)
        outs = list(it)

        g = g_ref[...]
        iv = infl_ref[...]
        accs = [None] * KP
        for k in range(KN):
            gk = g[:, k * m:(k + 1) * m]
            for p in range(KP):
                w = iv[:, k * KP + p:k * KP + p + 1]
                t = w * gk
                accs[p] = t if accs[p] is None else accs[p] + t
        s = jnp.concatenate(accs, axis=1)
        z = _leaky(_dot(s, wf_ref[...]) + bk_ref[...])
        o = _dot(z, wu_ref[...]) + bu_ref[...]
        x = x_ref[...]
        sc = _dot(x, wsc_ref[...]) if has_sc else x
        xo = _leaky(o + sc)
        if head is not None:
            outs[0][...] = _dot(xo, wh_ref[...]) + bh_ref[...]
            return
        outs[0][...] = xo
        if dn is not None:
            outs[1][...] = _leaky(_dot(xo, wdn_ref[...]) + bdn_ref[...])

    wspec = lambda i: (0, 0)
    in_specs = [pl.BlockSpec((blk, KN * m), lambda i: (i, 0)),
                pl.BlockSpec((blk, KN * KP), lambda i: (i, 0)),
                pl.BlockSpec((blk, ci), lambda i: (i, 0)),
                pl.BlockSpec((KP * m, m), wspec),
                pl.BlockSpec((1, m), wspec),
                pl.BlockSpec((m, co), wspec),
                pl.BlockSpec((1, co), wspec)]
    args = [gy, infl, x_in, wf, bk, wu, bu]
    if has_sc:
        in_specs.append(pl.BlockSpec((ci, co), wspec))
        args.append(wsc)
    if dn is not None:
        mn = wdn.shape[1]
        in_specs += [pl.BlockSpec((co, mn), wspec), pl.BlockSpec((1, mn), wspec)]
        args += [wdn, bdn]
    if head is not None:
        ch = wh.shape[1]
        in_specs += [pl.BlockSpec((co, ch), wspec), pl.BlockSpec((1, ch), wspec)]
        args += [wh, bh]

    if head is not None:
        out_specs = [pl.BlockSpec((blk, wh.shape[1]), lambda i: (i, 0))]
        out_shape = [jax.ShapeDtypeStruct((n, wh.shape[1]), F32)]
    else:
        out_specs = [pl.BlockSpec((blk, co), lambda i: (i, 0))]
        out_shape = [jax.ShapeDtypeStruct((n, co), F32)]
        if dn is not None:
            mn = wdn.shape[1]
            out_specs.append(pl.BlockSpec((blk, mn), lambda i: (i, 0)))
            out_shape.append(jax.ShapeDtypeStruct((n, mn), F32))

    res = pl.pallas_call(body, grid=(n // blk,), in_specs=in_specs,
                         out_specs=out_specs, out_shape=out_shape)(*args)
    return res


def _fold_lin(w, bn):
    return w * bn["g"][None, :], bn["b"].reshape(1, -1)


def _fold_kp(kpw, bn):
    pm, m = kpw.shape[0] * kpw.shape[1], kpw.shape[2]
    return kpw.reshape(pm, m) * bn["g"][None, :], bn["b"].reshape(1, -1)


def _rb_weights(p):
    wd, bd = _fold_lin(p["down"], p["bnd"])
    wf, bk = _fold_kp(p["kp"], p["bnk"])
    wu, bu = _fold_lin(p["up"], p["bnu"])
    wsc = p.get("sc")
    return dict(wd=wd, bd=bd, wf=wf, bk=bk, wu=wu, bu=bu, wsc=wsc)


def kernel(points, features, idx0, idx1, idx2, idx3, pool1, pool2, pool3,
           pos1, pos2, pos3, params):
    n0, n1, n2, n3 = points.shape[0], pos1.shape[0], pos2.shape[0], pos3.shape[0]
    np0, np1, np2, np3 = 10240, 2560, 640, 160
    b0, b1, b2, b3 = 256, 256, 128, 32

    def padw(x, w=16):
        return jnp.pad(x, ((0, 0), (0, w - x.shape[1])))

    t0 = _pad_rows(padw(jnp.concatenate([points, features], axis=1)), np0)
    p1 = _pad_rows(padw(pos1), np1)
    p2 = _pad_rows(padw(pos2), np2)
    p3 = _pad_rows(padw(pos3), np3)
    idx0f = _pad_rows(idx0, np0).reshape(-1)
    idx1f = _pad_rows(idx1, np1).reshape(-1)
    idx2f = _pad_rows(idx2, np2).reshape(-1)
    idx3f = _pad_rows(idx3, np3).reshape(-1)
    pool1f = _pad_rows(pool1, np1).reshape(-1)
    pool2f = _pad_rows(pool2, np2).reshape(-1)
    pool3f = _pad_rows(pool3, np3).reshape(-1)

    kpts = params["kpts"]
    w0, bb0 = _fold_kp(params["b1_kp"].reshape(KP, 1, 64), params["b1_bn"])
    rb_b1 = _rb_weights(params["b1_rb"])
    rb_a1 = _rb_weights(params["b1_ra"])
    rbs2 = [_rb_weights(p) for p in params["b2"]]
    rbs3 = [_rb_weights(p) for p in params["b3"]]
    rbs4 = [_rb_weights(p) for p in params["b4"]]
    head = (params["head_w"], params["head_b"].reshape(1, -1))

    g0 = _gather_rows(t0, idx0f)
    infl0, x0, y = _tc_level0(g0, t0, kpts[0], w0, bb0,
                              rb_b1["wd"], rb_b1["bd"], b0)
    gy = _gather_rows(y, idx0f)
    x1, y = _tc_resblock(gy, infl0, x0, rb_b1["wf"], rb_b1["bk"], rb_b1["wu"],
                         rb_b1["bu"], rb_b1["wsc"],
                         (rb_a1["wd"], rb_a1["bd"]), None, b0)
    gy = _gather_rows(y, idx0f)
    (x2,) = _tc_resblock(gy, infl0, x1, rb_a1["wf"], rb_a1["bk"], rb_a1["wu"],
                         rb_a1["bu"], rb_a1["wsc"], None, None, b0)
    skip1 = x2[:n0]

    x = x2
    skips = []
    level_cfg = [
        (p1, idx1f, pool1f, kpts[1], RADII[1], rbs2, np1, b1, n1),
        (p2, idx2f, pool2f, kpts[2], RADII[2], rbs3, np2, b2, n2),
        (p3, idx3f, pool3f, kpts[3], RADII[3], rbs4, np3, b3, n3),
    ]
    out = None
    for li, (pp, idxf, poolf, kp, rad, rbs, npl, blk, nl) in enumerate(level_cfg):
        gpos = _gather_rows(pp, idxf)
        infl = _tc_infl(gpos, pp, kp, rad, blk)
        gp = _gather_rows(x, poolf)
        x, y = _tc_pooldown(gp, rbs[0]["wd"], rbs[0]["bd"], blk)
        last = li == len(level_cfg) - 1
        for ri, rb in enumerate(rbs):
            gy = _gather_rows(y, idxf)
            final = ri == len(rbs) - 1
            nxt = None if final else (rbs[ri + 1]["wd"], rbs[ri + 1]["bd"])
            hd = head if (last and final) else None
            res = _tc_resblock(gy, infl, x, rb["wf"], rb["bk"], rb["wu"],
                               rb["bu"], rb["wsc"], nxt, hd, blk)
            if last and final:
                out = res[0]
            elif final:
                x = res[0]
            else:
                x, y = res
        if not last:
            skips.append(x[:nl])

    skip2, skip3 = skips
    return (out[:n3], skip1, skip2, skip3)


# speedup vs baseline: 1.0469x; 1.0469x over previous
import functools

import jax
import jax.numpy as jnp
from jax import lax
from jax.experimental import pallas as pl
from jax.experimental.pallas import tpu as pltpu
from jax.experimental.pallas import tpu_sc as plsc

KP = 15
KN = 16
NW = 32
RADII = (0.0625, 0.125, 0.25, 0.5)
F32 = jnp.float32


def _leaky(x):
    return jnp.where(x >= 0.0, x, 0.1 * x)


def _pad_rows(x, n):
    if x.shape[0] == n:
        return x
    return jnp.pad(x, ((0, n - x.shape[0]),) + ((0, 0),) * (x.ndim - 1))


def _sc_gather(table, idx_flat):
    _, D = table.shape
    B = idx_flat.shape[0]
    bpw = B // NW
    ch = min(128, bpw)
    while bpw % ch:
        ch -= 8
    g = max(1, min(45056 // (ch * D), 4))
    while bpw % (ch * g):
        g -= 1
    sch = ch * g
    nsch = bpw // sch
    mesh = plsc.VectorSubcoreMesh(core_axis_name="c", subcore_axis_name="s")

    @functools.partial(
        pl.kernel, mesh=mesh,
        out_type=jax.ShapeDtypeStruct((B, D), F32),
        compiler_params=pltpu.CompilerParams(use_tc_tiling_on_sc=False),
        scratch_types=[pltpu.VMEM((bpw,), jnp.int32),
                       pltpu.VMEM((sch, D), F32),
                       pltpu.VMEM((sch, D), F32),
                       pltpu.SemaphoreType.DMA,
                       pltpu.SemaphoreType.DMA,
                       pltpu.SemaphoreType.DMA,
                       pltpu.SemaphoreType.DMA])
    def gk(table_hbm, idx_hbm, out_hbm, idx_v, buf_a, buf_b, gs_a, gs_b,
           ws_a, ws_b):
        wid = lax.axis_index("s") * 2 + lax.axis_index("c")
        base = wid * bpw
        pltpu.sync_copy(idx_hbm.at[pl.ds(base, bpw)], idx_v)

        def fire(s, buf, gsem):
            for j in range(g):
                pltpu.async_copy(
                    table_hbm.at[idx_v.at[pl.ds(s * sch + j * ch, ch)]],
                    buf.at[pl.ds(j * ch, ch)], gsem)

        def retire(s, buf, gsem, wsem):
            pltpu.make_async_copy(table_hbm.at[pl.ds(0, sch)], buf, gsem).wait()
            pltpu.async_copy(buf, out_hbm.at[pl.ds(base + s * sch, sch)],
                             wsem).wait()

        fire(0, buf_a, gs_a)
        if nsch == 1:
            retire(0, buf_a, gs_a, ws_a)
        else:
            @pl.loop(0, nsch // 2)
            def _(h):
                s0 = 2 * h
                fire(s0 + 1, buf_b, gs_b)
                retire(s0, buf_a, gs_a, ws_a)

                @pl.when(s0 + 2 < nsch)
                def _():
                    fire(s0 + 2, buf_a, gs_a)
                retire(s0 + 1, buf_b, gs_b, ws_b)
            if nsch % 2:
                retire(nsch - 1, buf_a, gs_a, ws_a)

    return gk(table, idx_flat)


def _gather_rows(table, idx_flat):
    out = _sc_gather(table, idx_flat)
    return out.reshape(idx_flat.shape[0] // KN, KN * table.shape[1])


def _dot(a, b):
    return jax.lax.dot_general(a, b, (((1,), (0,)), ((), ())),
                               preferred_element_type=F32)


def _infl_cols(g, px, py, pz, kx, ky, kz, inv_r):
    cols = []
    for k in range(KN):
        o = 16 * k
        rx = g[:, o:o + 1] - px
        ry = g[:, o + 1:o + 2] - py
        rz = g[:, o + 2:o + 3] - pz
        dx = rx - kx
        dy = ry - ky
        dz = rz - kz
        d = jnp.sqrt(dx * dx + dy * dy + dz * dz + 1e-12)
        cols.append(jnp.maximum(0.0, 1.0 - d * inv_r))
    return cols


def _tc_level0(g0, t0, kpts, w0, b0, wd, bd, blk):
    n = t0.shape[0]
    inv_r = 1.0 / RADII[0]
    c0, m = w0.shape[1], wd.shape[1]
    kx, ky, kz = (kpts[:, i].reshape(1, KP) for i in range(3))

    def body(g_ref, t_ref, kx_ref, ky_ref, kz_ref, w0_ref, b0_ref,
             wd_ref, bd_ref, infl_ref, x0_ref, y1_ref):
        g = g_ref[...]
        t = t_ref[...]
        px, py, pz = t[:, 0:1], t[:, 1:2], t[:, 2:3]
        cols = _infl_cols(g, px, py, pz, kx_ref[...], ky_ref[...],
                          kz_ref[...], inv_r)
        agg = None
        for k in range(KN):
            a = cols[k] * g[:, 16 * k + 3:16 * k + 4]
            agg = a if agg is None else agg + a
        infl_ref[...] = jnp.concatenate(cols, axis=1)
        x0 = _leaky(_dot(agg, w0_ref[...]) + b0_ref[...])
        x0_ref[...] = x0
        y1_ref[...] = _leaky(_dot(x0, wd_ref[...]) + bd_ref[...])

    wspec = lambda i: (0, 0)
    return pl.pallas_call(
        body,
        grid=(n // blk,),
        in_specs=[pl.BlockSpec((blk, KN * 16), lambda i: (i, 0)),
                  pl.BlockSpec((blk, 16), lambda i: (i, 0)),
                  pl.BlockSpec((1, KP), wspec),
                  pl.BlockSpec((1, KP), wspec),
                  pl.BlockSpec((1, KP), wspec),
                  pl.BlockSpec((KP, c0), wspec),
                  pl.BlockSpec((1, c0), wspec),
                  pl.BlockSpec((c0, m), wspec),
                  pl.BlockSpec((1, m), wspec)],
        out_specs=[pl.BlockSpec((blk, KN * KP), lambda i: (i, 0)),
                   pl.BlockSpec((blk, c0), lambda i: (i, 0)),
                   pl.BlockSpec((blk, m), lambda i: (i, 0))],
        out_shape=[jax.ShapeDtypeStruct((n, KN * KP), F32),
                   jax.ShapeDtypeStruct((n, c0), F32),
                   jax.ShapeDtypeStruct((n, m), F32)],
    )(g0, t0, kx, ky, kz, w0, b0, wd, bd)


def _tc_infl(gpos, posp, kpts, radius, blk):
    n = posp.shape[0]
    inv_r = 1.0 / radius
    kx, ky, kz = (kpts[:, i].reshape(1, KP) for i in range(3))

    def body(g_ref, t_ref, kx_ref, ky_ref, kz_ref, infl_ref):
        g = g_ref[...]
        t = t_ref[...]
        cols = _infl_cols(g, t[:, 0:1], t[:, 1:2], t[:, 2:3],
                          kx_ref[...], ky_ref[...], kz_ref[...], inv_r)
        infl_ref[...] = jnp.concatenate(cols, axis=1)

    wspec = lambda i: (0, 0)
    return pl.pallas_call(
        body,
        grid=(n // blk,),
        in_specs=[pl.BlockSpec((blk, KN * 16), lambda i: (i, 0)),
                  pl.BlockSpec((blk, 16), lambda i: (i, 0)),
                  pl.BlockSpec((1, KP), wspec),
                  pl.BlockSpec((1, KP), wspec),
                  pl.BlockSpec((1, KP), wspec)],
        out_specs=pl.BlockSpec((blk, KN * KP), lambda i: (i, 0)),
        out_shape=jax.ShapeDtypeStruct((n, KN * KP), F32),
    )(gpos, posp, kx, ky, kz)


def _tc_pooldown(gp, wd, bd, blk):
    n = gp.shape[0]
    cp, m = wd.shape

    def body(g_ref, wd_ref, bd_ref, xp_ref, y_ref):
        g = g_ref[...]
        xp = g[:, :cp]
        for k in range(1, KN):
            xp = jnp.maximum(xp, g[:, k * cp:(k + 1) * cp])
        xp_ref[...] = xp
        y_ref[...] = _leaky(_dot(xp, wd_ref[...]) + bd_ref[...])

    wspec = lambda i: (0, 0)
    return pl.pallas_call(
        body,
        grid=(n // blk,),
        in_specs=[pl.BlockSpec((blk, KN * cp), lambda i: (i, 0)),
                  pl.BlockSpec((cp, m), wspec),
                  pl.BlockSpec((1, m), wspec)],
        out_specs=[pl.BlockSpec((blk, cp), lambda i: (i, 0)),
                   pl.BlockSpec((blk, m), lambda i: (i, 0))],
        out_shape=[jax.ShapeDtypeStruct((n, cp), F32),
                   jax.ShapeDtypeStruct((n, m), F32)],
    )(gp, wd, bd)


def _tc_resblock(gy, infl, x_in, wf, bk, wu, bu, wsc, dn, head, blk):
    n = gy.shape[0]
    m = wf.shape[1]
    ci = x_in.shape[1]
    co = wu.shape[1]
    has_sc = wsc is not None
    wdn, bdn = dn if dn is not None else (None, None)
    wh, bh = head if head is not None else (None, None)

    def body(*refs):
        it = iter(refs)
        g_ref, infl_ref, x_ref, wf_ref, bk_ref, wu_ref, bu_ref = (
            next(it) for _ in range(7))
        wsc_ref = next(it) if has_sc else None
        wdn_ref, bdn_ref = (next(it), next(it)) if dn is not None else (None, None)
        wh_ref, bh_ref = (next(it), next(it)) if head is not None else (None, None)
        outs = list(it)

        g = g_ref[...]
        iv = infl_ref[...]
        accs = [None] * KP
        for k in range(KN):
            gk = g[:, k * m:(k + 1) * m]
            for p in range(KP):
                w = iv[:, k * KP + p:k * KP + p + 1]
                t = w * gk
                accs[p] = t if accs[p] is None else accs[p] + t
        s = jnp.concatenate(accs, axis=1)
        z = _leaky(_dot(s, wf_ref[...]) + bk_ref[...])
        o = _dot(z, wu_ref[...]) + bu_ref[...]
        x = x_ref[...]
        sc = _dot(x, wsc_ref[...]) if has_sc else x
        xo = _leaky(o + sc)
        if head is not None:
            outs[0][...] = _dot(xo, wh_ref[...]) + bh_ref[...]
            return
        outs[0][...] = xo
        if dn is not None:
            outs[1][...] = _leaky(_dot(xo, wdn_ref[...]) + bdn_ref[...])

    wspec = lambda i: (0, 0)
    in_specs = [pl.BlockSpec((blk, KN * m), lambda i: (i, 0)),
                pl.BlockSpec((blk, KN * KP), lambda i: (i, 0)),
                pl.BlockSpec((blk, ci), lambda i: (i, 0)),
                pl.BlockSpec((KP * m, m), wspec),
                pl.BlockSpec((1, m), wspec),
                pl.BlockSpec((m, co), wspec),
                pl.BlockSpec((1, co), wspec)]
    args = [gy, infl, x_in, wf, bk, wu, bu]
    if has_sc:
        in_specs.append(pl.BlockSpec((ci, co), wspec))
        args.append(wsc)
    if dn is not None:
        mn = wdn.shape[1]
        in_specs += [pl.BlockSpec((co, mn), wspec), pl.BlockSpec((1, mn), wspec)]
        args += [wdn, bdn]
    if head is not None:
        ch = wh.shape[1]
        in_specs += [pl.BlockSpec((co, ch), wspec), pl.BlockSpec((1, ch), wspec)]
        args += [wh, bh]

    if head is not None:
        out_specs = [pl.BlockSpec((blk, wh.shape[1]), lambda i: (i, 0))]
        out_shape = [jax.ShapeDtypeStruct((n, wh.shape[1]), F32)]
    else:
        out_specs = [pl.BlockSpec((blk, co), lambda i: (i, 0))]
        out_shape = [jax.ShapeDtypeStruct((n, co), F32)]
        if dn is not None:
            mn = wdn.shape[1]
            out_specs.append(pl.BlockSpec((blk, mn), lambda i: (i, 0)))
            out_shape.append(jax.ShapeDtypeStruct((n, mn), F32))

    res = pl.pallas_call(body, grid=(n // blk,), in_specs=in_specs,
                         out_specs=out_specs, out_shape=out_shape)(*args)
    return res


def _fold_lin(w, bn):
    return w * bn["g"][None, :], bn["b"].reshape(1, -1)


def _fold_kp(kpw, bn):
    pm, m = kpw.shape[0] * kpw.shape[1], kpw.shape[2]
    return kpw.reshape(pm, m) * bn["g"][None, :], bn["b"].reshape(1, -1)


def _rb_weights(p):
    wd, bd = _fold_lin(p["down"], p["bnd"])
    wf, bk = _fold_kp(p["kp"], p["bnk"])
    wu, bu = _fold_lin(p["up"], p["bnu"])
    wsc = p.get("sc")
    return dict(wd=wd, bd=bd, wf=wf, bk=bk, wu=wu, bu=bu, wsc=wsc)


def kernel(points, features, idx0, idx1, idx2, idx3, pool1, pool2, pool3,
           pos1, pos2, pos3, params):
    n0, n1, n2, n3 = points.shape[0], pos1.shape[0], pos2.shape[0], pos3.shape[0]
    np0, np1, np2, np3 = 10240, 2560, 640, 160
    b0, b1, b2, b3 = 256, 256, 128, 32

    def padw(x, w=16):
        return jnp.pad(x, ((0, 0), (0, w - x.shape[1])))

    t0 = _pad_rows(padw(jnp.concatenate([points, features], axis=1)), np0)
    p1 = _pad_rows(padw(pos1), np1)
    p2 = _pad_rows(padw(pos2), np2)
    p3 = _pad_rows(padw(pos3), np3)
    idx0f = _pad_rows(idx0, np0).reshape(-1)
    idx1f = _pad_rows(idx1, np1).reshape(-1)
    idx2f = _pad_rows(idx2, np2).reshape(-1)
    idx3f = _pad_rows(idx3, np3).reshape(-1)
    pool1f = _pad_rows(pool1, np1).reshape(-1)
    pool2f = _pad_rows(pool2, np2).reshape(-1)
    pool3f = _pad_rows(pool3, np3).reshape(-1)

    kpts = params["kpts"]
    w0, bb0 = _fold_kp(params["b1_kp"].reshape(KP, 1, 64), params["b1_bn"])
    rb_b1 = _rb_weights(params["b1_rb"])
    rb_a1 = _rb_weights(params["b1_ra"])
    rbs2 = [_rb_weights(p) for p in params["b2"]]
    rbs3 = [_rb_weights(p) for p in params["b3"]]
    rbs4 = [_rb_weights(p) for p in params["b4"]]
    head = (params["head_w"], params["head_b"].reshape(1, -1))

    g0 = _gather_rows(t0, idx0f)
    infl0, x0, y = _tc_level0(g0, t0, kpts[0], w0, bb0,
                              rb_b1["wd"], rb_b1["bd"], b0)
    gy = _gather_rows(y, idx0f)
    x1, y = _tc_resblock(gy, infl0, x0, rb_b1["wf"], rb_b1["bk"], rb_b1["wu"],
                         rb_b1["bu"], rb_b1["wsc"],
                         (rb_a1["wd"], rb_a1["bd"]), None, b0)
    gy = _gather_rows(y, idx0f)
    (x2,) = _tc_resblock(gy, infl0, x1, rb_a1["wf"], rb_a1["bk"], rb_a1["wu"],
                         rb_a1["bu"], rb_a1["wsc"], None, None, b0)
    skip1 = x2[:n0]

    x = x2
    skips = []
    level_cfg = [
        (p1, idx1f, pool1f, kpts[1], RADII[1], rbs2, np1, b1, n1),
        (p2, idx2f, pool2f, kpts[2], RADII[2], rbs3, np2, b2, n2),
        (p3, idx3f, pool3f, kpts[3], RADII[3], rbs4, np3, b3, n3),
    ]
    out = None
    for li, (pp, idxf, poolf, kp, rad, rbs, npl, blk, nl) in enumerate(level_cfg):
        gpos = _gather_rows(pp, idxf)
        infl = _tc_infl(gpos, pp, kp, rad, blk)
        gp = _gather_rows(x, poolf)
        x, y = _tc_pooldown(gp, rbs[0]["wd"], rbs[0]["bd"], blk)
        last = li == len(level_cfg) - 1
        for ri, rb in enumerate(rbs):
            gy = _gather_rows(y, idxf)
            final = ri == len(rbs) - 1
            nxt = None if final else (rbs[ri + 1]["wd"], rbs[ri + 1]["bd"])
            hd = head if (last and final) else None
            res = _tc_resblock(gy, infl, x, rb["wf"], rb["bk"], rb["wu"],
                               rb["bu"], rb["wsc"], nxt, hd, blk)
            if last and final:
                out = res[0]
            elif final:
                x = res[0]
            else:
                x, y = res
        if not last:
            skips.append(x[:nl])

    skip2, skip3 = skips
    return (out[:n3], skip1, skip2, skip3)


# speedup vs baseline: 3.3331x; 3.1839x over previous
import functools

import jax
import jax.numpy as jnp
from jax import lax
from jax.experimental import pallas as pl
from jax.experimental.pallas import tpu as pltpu
from jax.experimental.pallas import tpu_sc as plsc

KP = 15
KN = 16
NW = 32
RADII = (0.0625, 0.125, 0.25, 0.5)
F32 = jnp.float32


def _leaky(x):
    return jnp.where(x >= 0.0, x, 0.1 * x)


def _pad_rows(x, n):
    if x.shape[0] == n:
        return x
    return jnp.pad(x, ((0, n - x.shape[0]),) + ((0, 0),) * (x.ndim - 1))


def _sc_gather(table, idx_flat):
    _, D = table.shape
    B = idx_flat.shape[0]
    bpw = B // NW
    ch = min(128, bpw)
    while bpw % ch:
        ch -= 8
    g = max(1, min(45056 // (ch * D), 4))
    while bpw % (ch * g):
        g -= 1
    sch = ch * g
    nsch = bpw // sch
    mesh = plsc.VectorSubcoreMesh(core_axis_name="c", subcore_axis_name="s")

    @functools.partial(
        pl.kernel, mesh=mesh,
        out_type=jax.ShapeDtypeStruct((B, D), F32),
        compiler_params=pltpu.CompilerParams(use_tc_tiling_on_sc=False),
        scratch_types=[pltpu.VMEM((bpw,), jnp.int32),
                       pltpu.VMEM((sch, D), F32),
                       pltpu.VMEM((sch, D), F32),
                       pltpu.SemaphoreType.DMA,
                       pltpu.SemaphoreType.DMA,
                       pltpu.SemaphoreType.DMA,
                       pltpu.SemaphoreType.DMA])
    def gk(table_hbm, idx_hbm, out_hbm, idx_v, buf_a, buf_b, gs_a, gs_b,
           ws_a, ws_b):
        wid = lax.axis_index("s") * 2 + lax.axis_index("c")
        base = wid * bpw
        pltpu.sync_copy(idx_hbm.at[pl.ds(base, bpw)], idx_v)

        def fire(s, buf, gsem):
            for j in range(g):
                pltpu.async_copy(
                    table_hbm.at[idx_v.at[pl.ds(s * sch + j * ch, ch)]],
                    buf.at[pl.ds(j * ch, ch)], gsem)

        def retire(s, buf, gsem, wsem):
            pltpu.make_async_copy(table_hbm.at[pl.ds(0, sch)], buf, gsem).wait()
            pltpu.async_copy(buf, out_hbm.at[pl.ds(base + s * sch, sch)],
                             wsem).wait()

        fire(0, buf_a, gs_a)
        if nsch == 1:
            retire(0, buf_a, gs_a, ws_a)
        else:
            @pl.loop(0, nsch // 2)
            def _(h):
                s0 = 2 * h
                fire(s0 + 1, buf_b, gs_b)
                retire(s0, buf_a, gs_a, ws_a)

                @pl.when(s0 + 2 < nsch)
                def _():
                    fire(s0 + 2, buf_a, gs_a)
                retire(s0 + 1, buf_b, gs_b, ws_b)
            if nsch % 2:
                retire(nsch - 1, buf_a, gs_a, ws_a)

    return gk(table, idx_flat)


def _gather_rows(table, idx_flat):
    out = _sc_gather(table, idx_flat)
    return out.reshape(idx_flat.shape[0] // KN, KN * table.shape[1])


def _dot(a, b):
    return jax.lax.dot_general(a, b, (((1,), (0,)), ((), ())),
                               preferred_element_type=F32)


def _infl_cols(g, px, py, pz, kx, ky, kz, inv_r):
    cols = []
    for k in range(KN):
        o = 16 * k
        rx = g[:, o:o + 1] - px
        ry = g[:, o + 1:o + 2] - py
        rz = g[:, o + 2:o + 3] - pz
        dx = rx - kx
        dy = ry - ky
        dz = rz - kz
        d = jnp.sqrt(dx * dx + dy * dy + dz * dz + 1e-12)
        cols.append(jnp.maximum(0.0, 1.0 - d * inv_r))
    return cols


def _tc_level0(g0, t0, kpts, w0, b0, wd, bd, blk):
    n = t0.shape[0]
    inv_r = 1.0 / RADII[0]
    c0, m = w0.shape[1], wd.shape[1]
    kx, ky, kz = (kpts[:, i].reshape(1, KP) for i in range(3))

    def body(g_ref, t_ref, kx_ref, ky_ref, kz_ref, w0_ref, b0_ref,
             wd_ref, bd_ref, infl_ref, x0_ref, y1_ref):
        g = g_ref[...]
        t = t_ref[...]
        px, py, pz = t[:, 0:1], t[:, 1:2], t[:, 2:3]
        cols = _infl_cols(g, px, py, pz, kx_ref[...], ky_ref[...],
                          kz_ref[...], inv_r)
        agg = None
        for k in range(KN):
            a = cols[k] * g[:, 16 * k + 3:16 * k + 4]
            agg = a if agg is None else agg + a
        infl_ref[...] = jnp.transpose(jnp.concatenate(cols, axis=1))
        x0 = _leaky(_dot(agg, w0_ref[...]) + b0_ref[...])
        x0_ref[...] = x0
        y1_ref[...] = _leaky(_dot(x0, wd_ref[...]) + bd_ref[...])

    wspec = lambda i: (0, 0)
    return pl.pallas_call(
        body,
        grid=(n // blk,),
        in_specs=[pl.BlockSpec((blk, KN * 16), lambda i: (i, 0)),
                  pl.BlockSpec((blk, 16), lambda i: (i, 0)),
                  pl.BlockSpec((1, KP), wspec),
                  pl.BlockSpec((1, KP), wspec),
                  pl.BlockSpec((1, KP), wspec),
                  pl.BlockSpec((KP, c0), wspec),
                  pl.BlockSpec((1, c0), wspec),
                  pl.BlockSpec((c0, m), wspec),
                  pl.BlockSpec((1, m), wspec)],
        out_specs=[pl.BlockSpec((KN * KP, blk), lambda i: (0, i)),
                   pl.BlockSpec((blk, c0), lambda i: (i, 0)),
                   pl.BlockSpec((blk, m), lambda i: (i, 0))],
        out_shape=[jax.ShapeDtypeStruct((KN * KP, n), F32),
                   jax.ShapeDtypeStruct((n, c0), F32),
                   jax.ShapeDtypeStruct((n, m), F32)],
    )(g0, t0, kx, ky, kz, w0, b0, wd, bd)


def _tc_infl(gpos, posp, kpts, radius, blk):
    n = posp.shape[0]
    inv_r = 1.0 / radius
    kx, ky, kz = (kpts[:, i].reshape(1, KP) for i in range(3))

    def body(g_ref, t_ref, kx_ref, ky_ref, kz_ref, infl_ref):
        g = g_ref[...]
        t = t_ref[...]
        cols = _infl_cols(g, t[:, 0:1], t[:, 1:2], t[:, 2:3],
                          kx_ref[...], ky_ref[...], kz_ref[...], inv_r)
        infl_ref[...] = jnp.transpose(jnp.concatenate(cols, axis=1))

    wspec = lambda i: (0, 0)
    return pl.pallas_call(
        body,
        grid=(n // blk,),
        in_specs=[pl.BlockSpec((blk, KN * 16), lambda i: (i, 0)),
                  pl.BlockSpec((blk, 16), lambda i: (i, 0)),
                  pl.BlockSpec((1, KP), wspec),
                  pl.BlockSpec((1, KP), wspec),
                  pl.BlockSpec((1, KP), wspec)],
        out_specs=pl.BlockSpec((KN * KP, blk), lambda i: (0, i)),
        out_shape=jax.ShapeDtypeStruct((KN * KP, n), F32),
    )(gpos, posp, kx, ky, kz)


def _tc_pooldown(gp, wd, bd, blk):
    n = gp.shape[0]
    cp, m = wd.shape

    def body(g_ref, wd_ref, bd_ref, xp_ref, y_ref):
        g = g_ref[...]
        xp = g[:, :cp]
        for k in range(1, KN):
            xp = jnp.maximum(xp, g[:, k * cp:(k + 1) * cp])
        xp_ref[...] = xp
        y_ref[...] = _leaky(_dot(xp, wd_ref[...]) + bd_ref[...])

    wspec = lambda i: (0, 0)
    return pl.pallas_call(
        body,
        grid=(n // blk,),
        in_specs=[pl.BlockSpec((blk, KN * cp), lambda i: (i, 0)),
                  pl.BlockSpec((cp, m), wspec),
                  pl.BlockSpec((1, m), wspec)],
        out_specs=[pl.BlockSpec((blk, cp), lambda i: (i, 0)),
                   pl.BlockSpec((blk, m), lambda i: (i, 0))],
        out_shape=[jax.ShapeDtypeStruct((n, cp), F32),
                   jax.ShapeDtypeStruct((n, m), F32)],
    )(gp, wd, bd)


def _tc_resblock(gy, infl, x_in, wf, bk, wu, bu, wsc, dn, head, blk):
    n = gy.shape[0]
    m = wf.shape[1]
    ci = x_in.shape[1]
    co = wu.shape[1]
    has_sc = wsc is not None
    wdn, bdn = dn if dn is not None else (None, None)
    wh, bh = head if head is not None else (None, None)

    def body(*refs):
        it = iter(refs)
        g_ref, infl_ref, x_ref, wf_ref, bk_ref, wu_ref, bu_ref = (
            next(it) for _ in range(7))
        wsc_ref = next(it) if has_sc else None
        wdn_ref, bdn_ref = (next(it), next(it)) if dn is not None else (None, None)
        wh_ref, bh_ref = (next(it), next(it)) if head is not None else (None, None)
        outs = list(it)

        gt = jnp.transpose(g_ref[...])
        iv = infl_ref[...]
        accs = [None] * KP
        for k in range(KN):
            gk = gt[k * m:(k + 1) * m, :]
            for p in range(KP):
                w = iv[k * KP + p:k * KP + p + 1, :]
                t = w * gk
                accs[p] = t if accs[p] is None else accs[p] + t
        s = jnp.concatenate(accs, axis=0)
        z = _leaky(_dot(wf_ref[...], s) + bk_ref[...])
        o = _dot(wu_ref[...], z) + bu_ref[...]
        xt = jnp.transpose(x_ref[...])
        sc = _dot(wsc_ref[...], xt) if has_sc else xt
        xo = _leaky(o + sc)
        if head is not None:
            outs[0][...] = jnp.transpose(_dot(wh_ref[...], xo) + bh_ref[...])
            return
        outs[0][...] = jnp.transpose(xo)
        if dn is not None:
            outs[1][...] = jnp.transpose(
                _leaky(_dot(wdn_ref[...], xo) + bdn_ref[...]))

    wspec = lambda i: (0, 0)
    in_specs = [pl.BlockSpec((blk, KN * m), lambda i: (i, 0)),
                pl.BlockSpec((KN * KP, blk), lambda i: (0, i)),
                pl.BlockSpec((blk, ci), lambda i: (i, 0)),
                pl.BlockSpec((m, KP * m), wspec),
                pl.BlockSpec((m, 1), wspec),
                pl.BlockSpec((co, m), wspec),
                pl.BlockSpec((co, 1), wspec)]
    args = [gy, infl, x_in, wf.T, bk.T, wu.T, bu.T]
    if has_sc:
        in_specs.append(pl.BlockSpec((co, ci), wspec))
        args.append(wsc.T)
    if dn is not None:
        mn = wdn.shape[1]
        in_specs += [pl.BlockSpec((mn, co), wspec), pl.BlockSpec((mn, 1), wspec)]
        args += [wdn.T, bdn.T]
    if head is not None:
        ch = wh.shape[1]
        in_specs += [pl.BlockSpec((ch, co), wspec), pl.BlockSpec((ch, 1), wspec)]
        args += [wh.T, bh.T]

    if head is not None:
        out_specs = [pl.BlockSpec((blk, wh.shape[1]), lambda i: (i, 0))]
        out_shape = [jax.ShapeDtypeStruct((n, wh.shape[1]), F32)]
    else:
        out_specs = [pl.BlockSpec((blk, co), lambda i: (i, 0))]
        out_shape = [jax.ShapeDtypeStruct((n, co), F32)]
        if dn is not None:
            mn = wdn.shape[1]
            out_specs.append(pl.BlockSpec((blk, mn), lambda i: (i, 0)))
            out_shape.append(jax.ShapeDtypeStruct((n, mn), F32))

    res = pl.pallas_call(body, grid=(n // blk,), in_specs=in_specs,
                         out_specs=out_specs, out_shape=out_shape)(*args)
    return res


def _fold_lin(w, bn):
    return w * bn["g"][None, :], bn["b"].reshape(1, -1)


def _fold_kp(kpw, bn):
    pm, m = kpw.shape[0] * kpw.shape[1], kpw.shape[2]
    return kpw.reshape(pm, m) * bn["g"][None, :], bn["b"].reshape(1, -1)


def _rb_weights(p):
    wd, bd = _fold_lin(p["down"], p["bnd"])
    wf, bk = _fold_kp(p["kp"], p["bnk"])
    wu, bu = _fold_lin(p["up"], p["bnu"])
    wsc = p.get("sc")
    return dict(wd=wd, bd=bd, wf=wf, bk=bk, wu=wu, bu=bu, wsc=wsc)


def kernel(points, features, idx0, idx1, idx2, idx3, pool1, pool2, pool3,
           pos1, pos2, pos3, params):
    n0, n1, n2, n3 = points.shape[0], pos1.shape[0], pos2.shape[0], pos3.shape[0]
    np0, np1, np2, np3 = 10240, 2560, 640, 160
    b0, b1, b2, b3 = 256, 256, 128, 160
    r0, r1, r2, r3 = 512, 512, 128, 160

    def padw(x, w=16):
        return jnp.pad(x, ((0, 0), (0, w - x.shape[1])))

    t0 = _pad_rows(padw(jnp.concatenate([points, features], axis=1)), np0)
    p1 = _pad_rows(padw(pos1), np1)
    p2 = _pad_rows(padw(pos2), np2)
    p3 = _pad_rows(padw(pos3), np3)
    idx0f = _pad_rows(idx0, np0).reshape(-1)
    idx1f = _pad_rows(idx1, np1).reshape(-1)
    idx2f = _pad_rows(idx2, np2).reshape(-1)
    idx3f = _pad_rows(idx3, np3).reshape(-1)
    pool1f = _pad_rows(pool1, np1).reshape(-1)
    pool2f = _pad_rows(pool2, np2).reshape(-1)
    pool3f = _pad_rows(pool3, np3).reshape(-1)

    kpts = params["kpts"]
    w0, bb0 = _fold_kp(params["b1_kp"].reshape(KP, 1, 64), params["b1_bn"])
    rb_b1 = _rb_weights(params["b1_rb"])
    rb_a1 = _rb_weights(params["b1_ra"])
    rbs2 = [_rb_weights(p) for p in params["b2"]]
    rbs3 = [_rb_weights(p) for p in params["b3"]]
    rbs4 = [_rb_weights(p) for p in params["b4"]]
    head = (params["head_w"], params["head_b"].reshape(1, -1))

    g0 = _gather_rows(t0, idx0f)
    infl0, x0, y = _tc_level0(g0, t0, kpts[0], w0, bb0,
                              rb_b1["wd"], rb_b1["bd"], b0)
    gy = _gather_rows(y, idx0f)
    x1, y = _tc_resblock(gy, infl0, x0, rb_b1["wf"], rb_b1["bk"], rb_b1["wu"],
                         rb_b1["bu"], rb_b1["wsc"],
                         (rb_a1["wd"], rb_a1["bd"]), None, r0)
    gy = _gather_rows(y, idx0f)
    (x2,) = _tc_resblock(gy, infl0, x1, rb_a1["wf"], rb_a1["bk"], rb_a1["wu"],
                         rb_a1["bu"], rb_a1["wsc"], None, None, r0)
    skip1 = x2[:n0]

    x = x2
    skips = []
    level_cfg = [
        (p1, idx1f, pool1f, kpts[1], RADII[1], rbs2, np1, b1, r1, n1),
        (p2, idx2f, pool2f, kpts[2], RADII[2], rbs3, np2, b2, r2, n2),
        (p3, idx3f, pool3f, kpts[3], RADII[3], rbs4, np3, b3, r3, n3),
    ]
    out = None
    for li, (pp, idxf, poolf, kp, rad, rbs, npl, blk, rblk, nl) in enumerate(level_cfg):
        gpos = _gather_rows(pp, idxf)
        infl = _tc_infl(gpos, pp, kp, rad, blk)
        gp = _gather_rows(x, poolf)
        x, y = _tc_pooldown(gp, rbs[0]["wd"], rbs[0]["bd"], blk)
        last = li == len(level_cfg) - 1
        for ri, rb in enumerate(rbs):
            gy = _gather_rows(y, idxf)
            final = ri == len(rbs) - 1
            nxt = None if final else (rbs[ri + 1]["wd"], rbs[ri + 1]["bd"])
            hd = head if (last and final) else None
            res = _tc_resblock(gy, infl, x, rb["wf"], rb["bk"], rb["wu"],
                               rb["bu"], rb["wsc"], nxt, hd, rblk)
            if last and final:
                out = res[0]
            elif final:
                x = res[0]
            else:
                x, y = res
        if not last:
            skips.append(x[:nl])

    skip2, skip3 = skips
    return (out[:n3], skip1, skip2, skip3)


# speedup vs baseline: 3.9822x; 1.1947x over previous
import functools

import jax
import jax.numpy as jnp
from jax import lax
from jax.experimental import pallas as pl
from jax.experimental.pallas import tpu as pltpu
from jax.experimental.pallas import tpu_sc as plsc

KP = 15
KN = 16
NW = 32
RADII = (0.0625, 0.125, 0.25, 0.5)
F32 = jnp.float32


def _leaky(x):
    return jnp.where(x >= 0.0, x, 0.1 * x)


def _pad_rows(x, n):
    if x.shape[0] == n:
        return x
    return jnp.pad(x, ((0, n - x.shape[0]),) + ((0, 0),) * (x.ndim - 1))


def _sc_plan(B, D):
    bpw = B // NW
    ch = min(128, bpw)
    while bpw % ch:
        ch -= 8
    g = max(1, min(45056 // (ch * D), 4))
    while bpw % (ch * g):
        g -= 1
    sch = ch * g
    return bpw, ch, g, sch, bpw // sch


def _sc_pipeline(table_hbm, idx_hbm, out_hbm, idx_v, buf_a, buf_b,
                 gs_a, gs_b, ws_a, ws_b, base, plan):
    bpw, ch, g, sch, nsch = plan
    pltpu.sync_copy(idx_hbm.at[pl.ds(base, bpw)], idx_v)

    def fire(s, buf, gsem):
        for j in range(g):
            pltpu.async_copy(
                table_hbm.at[idx_v.at[pl.ds(s * sch + j * ch, ch)]],
                buf.at[pl.ds(j * ch, ch)], gsem)

    def retire(s, buf, gsem, wsem):
        pltpu.make_async_copy(table_hbm.at[pl.ds(0, sch)], buf, gsem).wait()
        pltpu.async_copy(buf, out_hbm.at[pl.ds(base + s * sch, sch)],
                         wsem).wait()

    fire(0, buf_a, gs_a)
    if nsch == 1:
        retire(0, buf_a, gs_a, ws_a)
    else:
        @pl.loop(0, nsch // 2)
        def _(h):
            s0 = 2 * h
            fire(s0 + 1, buf_b, gs_b)
            retire(s0, buf_a, gs_a, ws_a)

            @pl.when(s0 + 2 < nsch)
            def _():
                fire(s0 + 2, buf_a, gs_a)
            retire(s0 + 1, buf_b, gs_b, ws_b)
        if nsch % 2:
            retire(nsch - 1, buf_a, gs_a, ws_a)


def _sc_gather_multi(pairs):
    plans = [_sc_plan(idx.shape[0], t.shape[1]) for t, idx in pairs]
    mesh = plsc.VectorSubcoreMesh(core_axis_name="c", subcore_axis_name="s")
    scratch = []
    for (t, idx), (bpw, ch, g, sch, nsch) in zip(pairs, plans):
        scratch += [pltpu.VMEM((bpw,), jnp.int32),
                    pltpu.VMEM((sch, t.shape[1]), F32),
                    pltpu.VMEM((sch, t.shape[1]), F32)]
    scratch += [pltpu.SemaphoreType.DMA] * 4

    @functools.partial(
        pl.kernel, mesh=mesh,
        out_type=[jax.ShapeDtypeStruct((idx.shape[0], t.shape[1]), F32)
                  for t, idx in pairs],
        compiler_params=pltpu.CompilerParams(use_tc_tiling_on_sc=False),
        scratch_types=scratch)
    def gk(*refs):
        n = len(pairs)
        tables = refs[:2 * n:2]
        idxs = refs[1:2 * n:2]
        outs = refs[2 * n:3 * n]
        rest = refs[3 * n:]
        sems = rest[-4:]
        wid = lax.axis_index("s") * 2 + lax.axis_index("c")
        for i, plan in enumerate(plans):
            _sc_pipeline(tables[i], idxs[i], outs[i],
                         rest[3 * i], rest[3 * i + 1], rest[3 * i + 2],
                         sems[0], sems[1], sems[2], sems[3],
                         wid * plan[0], plan)

    flat = []
    for t, idx in pairs:
        flat += [t, idx]
    res = gk(*flat)
    return res if isinstance(res, (list, tuple)) else [res]


def _sc_gather(table, idx_flat):
    _, D = table.shape
    B = idx_flat.shape[0]
    bpw = B // NW
    ch = min(128, bpw)
    while bpw % ch:
        ch -= 8
    g = max(1, min(45056 // (ch * D), 4))
    while bpw % (ch * g):
        g -= 1
    sch = ch * g
    nsch = bpw // sch
    mesh = plsc.VectorSubcoreMesh(core_axis_name="c", subcore_axis_name="s")

    @functools.partial(
        pl.kernel, mesh=mesh,
        out_type=jax.ShapeDtypeStruct((B, D), F32),
        compiler_params=pltpu.CompilerParams(use_tc_tiling_on_sc=False),
        scratch_types=[pltpu.VMEM((bpw,), jnp.int32),
                       pltpu.VMEM((sch, D), F32),
                       pltpu.VMEM((sch, D), F32),
                       pltpu.SemaphoreType.DMA,
                       pltpu.SemaphoreType.DMA,
                       pltpu.SemaphoreType.DMA,
                       pltpu.SemaphoreType.DMA])
    def gk(table_hbm, idx_hbm, out_hbm, idx_v, buf_a, buf_b, gs_a, gs_b,
           ws_a, ws_b):
        wid = lax.axis_index("s") * 2 + lax.axis_index("c")
        base = wid * bpw
        pltpu.sync_copy(idx_hbm.at[pl.ds(base, bpw)], idx_v)

        def fire(s, buf, gsem):
            for j in range(g):
                pltpu.async_copy(
                    table_hbm.at[idx_v.at[pl.ds(s * sch + j * ch, ch)]],
                    buf.at[pl.ds(j * ch, ch)], gsem)

        def retire(s, buf, gsem, wsem):
            pltpu.make_async_copy(table_hbm.at[pl.ds(0, sch)], buf, gsem).wait()
            pltpu.async_copy(buf, out_hbm.at[pl.ds(base + s * sch, sch)],
                             wsem).wait()

        fire(0, buf_a, gs_a)
        if nsch == 1:
            retire(0, buf_a, gs_a, ws_a)
        else:
            @pl.loop(0, nsch // 2)
            def _(h):
                s0 = 2 * h
                fire(s0 + 1, buf_b, gs_b)
                retire(s0, buf_a, gs_a, ws_a)

                @pl.when(s0 + 2 < nsch)
                def _():
                    fire(s0 + 2, buf_a, gs_a)
                retire(s0 + 1, buf_b, gs_b, ws_b)
            if nsch % 2:
                retire(nsch - 1, buf_a, gs_a, ws_a)

    return gk(table, idx_flat)


def _gather_rows(table, idx_flat):
    out = _sc_gather(table, idx_flat)
    return out.reshape(idx_flat.shape[0] // KN, KN * table.shape[1])


def _dot(a, b):
    return jax.lax.dot_general(a, b, (((1,), (0,)), ((), ())),
                               preferred_element_type=F32)


def _kp_cols(kpts):
    return tuple(kpts[:, i].reshape(KP, 1) for i in range(3))


def _infl_rows_t(gt, pxt, pyt, pzt, kxc, kyc, kzc, inv_r, stride=16):
    rows = []
    for k in range(KN):
        o = stride * k
        rx = gt[o:o + 1, :] - pxt
        ry = gt[o + 1:o + 2, :] - pyt
        rz = gt[o + 2:o + 3, :] - pzt
        dx = rx - kxc
        dy = ry - kyc
        dz = rz - kzc
        d = jnp.sqrt(dx * dx + dy * dy + dz * dz + 1e-12)
        rows.append(jnp.maximum(0.0, 1.0 - d * inv_r))
    return rows


def _onehot(idxk, v):
    iota = lax.broadcasted_iota(jnp.int32, (v, 1), 0)
    return jnp.where(idxk == iota, 1.0, 0.0).astype(F32)


def _wspec(shape):
    return pl.BlockSpec(shape, lambda i: tuple(0 for _ in shape))


def _cspec(shape):
    return pl.BlockSpec(shape, lambda i: (0, i))


def _rspec(shape):
    return pl.BlockSpec(shape, lambda i: (i, 0))


def _tc_level0(g0, tT, kpts, w0, b0, wd, bd, blk):
    n = tT.shape[1]
    inv_r = 1.0 / RADII[0]
    c0, m = w0.shape[1], wd.shape[1]
    kxc, kyc, kzc = _kp_cols(kpts)

    def body(g_ref, t_ref, kx_ref, ky_ref, kz_ref, w0_ref, b0_ref,
             wd_ref, bd_ref, infl_ref, x0_ref, y1_ref):
        gt = jnp.transpose(g_ref[...])
        t = t_ref[...]
        rows = _infl_rows_t(gt, t[0:1, :], t[1:2, :], t[2:3, :],
                            kx_ref[...], ky_ref[...], kz_ref[...], inv_r)
        agg = None
        for k in range(KN):
            a = rows[k] * gt[16 * k + 3:16 * k + 4, :]
            agg = a if agg is None else agg + a
        infl_ref[...] = jnp.concatenate(rows, axis=0)
        x0 = _leaky(_dot(w0_ref[...], agg) + b0_ref[...])
        x0_ref[...] = x0
        y1_ref[...] = jnp.transpose(
            _leaky(_dot(wd_ref[...], x0) + bd_ref[...]))

    return pl.pallas_call(
        body,
        grid=(n // blk,),
        in_specs=[_rspec((blk, KN * 16)), _cspec((16, blk)),
                  _wspec((KP, 1)), _wspec((KP, 1)), _wspec((KP, 1)),
                  _wspec((c0, KP)), _wspec((c0, 1)),
                  _wspec((m, c0)), _wspec((m, 1))],
        out_specs=[_cspec((KN * KP, blk)), _cspec((c0, blk)),
                   _rspec((blk, m))],
        out_shape=[jax.ShapeDtypeStruct((KN * KP, n), F32),
                   jax.ShapeDtypeStruct((c0, n), F32),
                   jax.ShapeDtypeStruct((n, m), F32)],
    )(g0, tT, kxc, kyc, kzc, w0.T, b0.T, wd.T, bd.T)


def _tc_infl(gpos, pT, kpts, radius, blk):
    n = pT.shape[1]
    inv_r = 1.0 / radius
    kxc, kyc, kzc = _kp_cols(kpts)

    def body(g_ref, t_ref, kx_ref, ky_ref, kz_ref, infl_ref):
        gt = jnp.transpose(g_ref[...])
        t = t_ref[...]
        rows = _infl_rows_t(gt, t[0:1, :], t[1:2, :], t[2:3, :],
                            kx_ref[...], ky_ref[...], kz_ref[...], inv_r)
        infl_ref[...] = jnp.concatenate(rows, axis=0)

    return pl.pallas_call(
        body,
        grid=(n // blk,),
        in_specs=[_rspec((blk, KN * 16)), _cspec((16, blk)),
                  _wspec((KP, 1)), _wspec((KP, 1)), _wspec((KP, 1))],
        out_specs=_cspec((KN * KP, blk)),
        out_shape=jax.ShapeDtypeStruct((KN * KP, n), F32),
    )(gpos, pT, kxc, kyc, kzc)


def _tc_infl_oh(pT, idxT, kpts, radius, blk):
    v = pT.shape[1]
    n = idxT.shape[1]
    inv_r = 1.0 / radius
    kxc, kyc, kzc = _kp_cols(kpts)

    def body(p_ref, pc_ref, i_ref, kx_ref, ky_ref, kz_ref, infl_ref):
        pt = p_ref[...][0:8, :]
        pc = pc_ref[...]
        idx = i_ref[...]
        pxt, pyt, pzt = pc[0:1, :], pc[1:2, :], pc[2:3, :]
        kxcv, kycv, kzcv = kx_ref[...], ky_ref[...], kz_ref[...]
        rows = []
        for k in range(KN):
            gk = _dot(pt, _onehot(idx[k:k + 1, :], v))
            rx = gk[0:1, :] - pxt
            ry = gk[1:2, :] - pyt
            rz = gk[2:3, :] - pzt
            dx = rx - kxcv
            dy = ry - kycv
            dz = rz - kzcv
            d = jnp.sqrt(dx * dx + dy * dy + dz * dz + 1e-12)
            rows.append(jnp.maximum(0.0, 1.0 - d * inv_r))
        infl_ref[...] = jnp.concatenate(rows, axis=0)

    return pl.pallas_call(
        body,
        grid=(n // blk,),
        in_specs=[_wspec((16, v)), _cspec((16, blk)), _cspec((KN, blk)),
                  _wspec((KP, 1)), _wspec((KP, 1)), _wspec((KP, 1))],
        out_specs=_cspec((KN * KP, blk)),
        out_shape=jax.ShapeDtypeStruct((KN * KP, n), F32),
    )(pT, pT, idxT, kxc, kyc, kzc)


def _tc_pooldown(gp, wd, bd, blk, y_rm):
    n = gp.shape[0]
    cp, m = wd.shape

    def body(g_ref, wd_ref, bd_ref, xp_ref, y_ref):
        g = g_ref[...]
        xp = g[:, :cp]
        for k in range(1, KN):
            xp = jnp.maximum(xp, g[:, k * cp:(k + 1) * cp])
        xpt = jnp.transpose(xp)
        xp_ref[...] = xpt
        yt = _leaky(_dot(wd_ref[...], xpt) + bd_ref[...])
        y_ref[...] = jnp.transpose(yt) if y_rm else yt

    y_spec = _rspec((blk, m)) if y_rm else _cspec((m, blk))
    y_shape = (n, m) if y_rm else (m, n)
    return pl.pallas_call(
        body,
        grid=(n // blk,),
        in_specs=[_rspec((blk, KN * cp)), _wspec((m, cp)), _wspec((m, 1))],
        out_specs=[_cspec((cp, blk)), y_spec],
        out_shape=[jax.ShapeDtypeStruct((cp, n), F32),
                   jax.ShapeDtypeStruct(y_shape, F32)],
    )(gp, wd.T, bd.T)


def _tc_pooldown_oh(xT, poolT, wd, bd, blk):
    cp, v = xT.shape
    n = poolT.shape[1]
    m = wd.shape[1]

    def body(x_ref, i_ref, wd_ref, bd_ref, xp_ref, y_ref):
        xt = x_ref[...]
        idx = i_ref[...]
        xp = None
        for k in range(KN):
            gk = _dot(xt, _onehot(idx[k:k + 1, :], v))
            xp = gk if xp is None else jnp.maximum(xp, gk)
        xp_ref[...] = xp
        y_ref[...] = _leaky(_dot(wd_ref[...], xp) + bd_ref[...])

    return pl.pallas_call(
        body,
        grid=(n // blk,),
        in_specs=[_wspec((cp, v)), _cspec((KN, blk)),
                  _wspec((m, cp)), _wspec((m, 1))],
        out_specs=[_cspec((cp, blk)), _cspec((m, blk))],
        out_shape=[jax.ShapeDtypeStruct((cp, n), F32),
                   jax.ShapeDtypeStruct((m, n), F32)],
    )(xT, poolT, wd.T, bd.T)


def _agg_tail(gk_of_k, iv, wf_ref, bk_ref, wu_ref, bu_ref, wsc_ref, xt):
    accs = [None] * KP
    for k in range(KN):
        gk = gk_of_k(k)
        for p in range(KP):
            w = iv[k * KP + p:k * KP + p + 1, :]
            t = w * gk
            accs[p] = t if accs[p] is None else accs[p] + t
    s = jnp.concatenate(accs, axis=0)
    z = _leaky(_dot(wf_ref[...], s) + bk_ref[...])
    o = _dot(wu_ref[...], z) + bu_ref[...]
    sc = _dot(wsc_ref[...], xt) if wsc_ref is not None else xt
    return _leaky(o + sc)


def _tc_resblock(gy, inflT, xT_in, wf, bk, wu, bu, wsc, dn, head, blk,
                 out_rm=False):
    n = gy.shape[0]
    m = wf.shape[1]
    ci = xT_in.shape[0]
    co = wu.shape[1]
    has_sc = wsc is not None
    wdn, bdn = dn if dn is not None else (None, None)
    wh, bh = head if head is not None else (None, None)

    def body(*refs):
        it = iter(refs)
        g_ref, infl_ref, x_ref, wf_ref, bk_ref, wu_ref, bu_ref = (
            next(it) for _ in range(7))
        wsc_ref = next(it) if has_sc else None
        wdn_ref, bdn_ref = (next(it), next(it)) if dn is not None else (0, 0)
        wh_ref, bh_ref = (next(it), next(it)) if head is not None else (0, 0)
        outs = list(it)

        gt = jnp.transpose(g_ref[...])
        xo = _agg_tail(lambda k: gt[k * m:(k + 1) * m, :], infl_ref[...],
                       wf_ref, bk_ref, wu_ref, bu_ref, wsc_ref, x_ref[...])
        if head is not None:
            outs[0][...] = jnp.transpose(_dot(wh_ref[...], xo) + bh_ref[...])
            return
        outs[0][...] = jnp.transpose(xo) if out_rm else xo
        if dn is not None:
            outs[1][...] = jnp.transpose(
                _leaky(_dot(wdn_ref[...], xo) + bdn_ref[...]))

    in_specs = [_rspec((blk, KN * m)), _cspec((KN * KP, blk)),
                _cspec((ci, blk)),
                _wspec((m, KP * m)), _wspec((m, 1)),
                _wspec((co, m)), _wspec((co, 1))]
    args = [gy, inflT, xT_in, wf.T, bk.T, wu.T, bu.T]
    if has_sc:
        in_specs.append(_wspec((co, ci)))
        args.append(wsc.T)
    if dn is not None:
        mn = wdn.shape[1]
        in_specs += [_wspec((mn, co)), _wspec((mn, 1))]
        args += [wdn.T, bdn.T]
    if head is not None:
        chh = wh.shape[1]
        in_specs += [_wspec((chh, co)), _wspec((chh, 1))]
        args += [wh.T, bh.T]

    if head is not None:
        out_specs = [_rspec((blk, wh.shape[1]))]
        out_shape = [jax.ShapeDtypeStruct((n, wh.shape[1]), F32)]
    else:
        out_specs = [_rspec((blk, co)) if out_rm else _cspec((co, blk))]
        out_shape = [jax.ShapeDtypeStruct((n, co) if out_rm else (co, n), F32)]
        if dn is not None:
            mn = wdn.shape[1]
            out_specs.append(_rspec((blk, mn)))
            out_shape.append(jax.ShapeDtypeStruct((n, mn), F32))

    return pl.pallas_call(body, grid=(n // blk,), in_specs=in_specs,
                          out_specs=out_specs, out_shape=out_shape)(*args)


def _tc_resblock_oh(yT, idxT, inflT, xT_in, wf, bk, wu, bu, wsc, dn, head,
                    blk, out_rm=False):
    m, v = yT.shape
    n = idxT.shape[1]
    ci = xT_in.shape[0]
    co = wu.shape[1]
    has_sc = wsc is not None
    wdn, bdn = dn if dn is not None else (None, None)
    wh, bh = head if head is not None else (None, None)

    def body(*refs):
        it = iter(refs)
        y_ref, i_ref, infl_ref, x_ref, wf_ref, bk_ref, wu_ref, bu_ref = (
            next(it) for _ in range(8))
        wsc_ref = next(it) if has_sc else None
        wdn_ref, bdn_ref = (next(it), next(it)) if dn is not None else (0, 0)
        wh_ref, bh_ref = (next(it), next(it)) if head is not None else (0, 0)
        outs = list(it)

        yt = y_ref[...]
        idx = i_ref[...]
        xo = _agg_tail(lambda k: _dot(yt, _onehot(idx[k:k + 1, :], v)),
                       infl_ref[...], wf_ref, bk_ref, wu_ref, bu_ref,
                       wsc_ref, x_ref[...])
        if head is not None:
            outs[0][...] = jnp.transpose(_dot(wh_ref[...], xo) + bh_ref[...])
            return
        outs[0][...] = xo
        i = 1
        if out_rm:
            outs[i][...] = jnp.transpose(xo)
            i += 1
        if dn is not None:
            outs[i][...] = _leaky(_dot(wdn_ref[...], xo) + bdn_ref[...])

    in_specs = [_wspec((m, v)), _cspec((KN, blk)), _cspec((KN * KP, blk)),
                _cspec((ci, blk)),
                _wspec((m, KP * m)), _wspec((m, 1)),
                _wspec((co, m)), _wspec((co, 1))]
    args = [yT, idxT, inflT, xT_in, wf.T, bk.T, wu.T, bu.T]
    if has_sc:
        in_specs.append(_wspec((co, ci)))
        args.append(wsc.T)
    if dn is not None:
        mn = wdn.shape[1]
        in_specs += [_wspec((mn, co)), _wspec((mn, 1))]
        args += [wdn.T, bdn.T]
    if head is not None:
        chh = wh.shape[1]
        in_specs += [_wspec((chh, co)), _wspec((chh, 1))]
        args += [wh.T, bh.T]

    if head is not None:
        out_specs = [_rspec((blk, wh.shape[1]))]
        out_shape = [jax.ShapeDtypeStruct((n, wh.shape[1]), F32)]
    else:
        out_specs = [_cspec((co, blk))]
        out_shape = [jax.ShapeDtypeStruct((co, n), F32)]
        if out_rm:
            out_specs.append(_rspec((blk, co)))
            out_shape.append(jax.ShapeDtypeStruct((n, co), F32))
        if dn is not None:
            mn = wdn.shape[1]
            out_specs.append(_cspec((mn, blk)))
            out_shape.append(jax.ShapeDtypeStruct((mn, n), F32))

    return pl.pallas_call(body, grid=(n // blk,), in_specs=in_specs,
                          out_specs=out_specs, out_shape=out_shape)(*args)


def _fold_lin(w, bn):
    return w * bn["g"][None, :], bn["b"].reshape(1, -1)


def _fold_kp(kpw, bn):
    pm, m = kpw.shape[0] * kpw.shape[1], kpw.shape[2]
    return kpw.reshape(pm, m) * bn["g"][None, :], bn["b"].reshape(1, -1)


def _rb_weights(p):
    wd, bd = _fold_lin(p["down"], p["bnd"])
    wf, bk = _fold_kp(p["kp"], p["bnk"])
    wu, bu = _fold_lin(p["up"], p["bnu"])
    return dict(wd=wd, bd=bd, wf=wf, bk=bk, wu=wu, bu=bu, wsc=p.get("sc"))


def kernel(points, features, idx0, idx1, idx2, idx3, pool1, pool2, pool3,
           pos1, pos2, pos3, params):
    n0, n1, n2, n3 = points.shape[0], pos1.shape[0], pos2.shape[0], pos3.shape[0]
    np0, np1, np2, np3 = 10240, 2560, 640, 160
    b0, b1, b2, b3 = 256, 256, 128, 160
    r0, r1, r2, r3 = 512, 512, 128, 160

    def padw(x, w=16):
        return jnp.pad(x, ((0, 0), (0, w - x.shape[1])))

    t0 = _pad_rows(padw(jnp.concatenate([points, features], axis=1)), np0)
    p1 = _pad_rows(padw(pos1), np1)
    p2 = _pad_rows(padw(pos2), np2)
    p3 = _pad_rows(padw(pos3), np3)
    t0T, p1T, p2T, p3T = t0.T, p1.T, p2.T, p3.T
    idx0f = _pad_rows(idx0, np0).reshape(-1)
    idx1f = _pad_rows(idx1, np1).reshape(-1)
    idx2T = _pad_rows(idx2, np2).T
    idx3T = idx3.T
    pool1f = _pad_rows(pool1, np1).reshape(-1)
    pool2f = _pad_rows(pool2, np2).reshape(-1)
    pool3T = pool3.T

    kpts = params["kpts"]
    w0, bb0 = _fold_kp(params["b1_kp"].reshape(KP, 1, 64), params["b1_bn"])
    rb_b1 = _rb_weights(params["b1_rb"])
    rb_a1 = _rb_weights(params["b1_ra"])
    rbs2 = [_rb_weights(p) for p in params["b2"]]
    rbs3 = [_rb_weights(p) for p in params["b3"]]
    rbs4 = [_rb_weights(p) for p in params["b4"]]
    head = (params["head_w"], params["head_b"].reshape(1, -1))

    g0r, gpos1r = _sc_gather_multi([(t0, idx0f), (p1, idx1f)])
    g0 = g0r.reshape(np0, KN * 16)
    gpos1 = gpos1r.reshape(np1, KN * 16)
    infl0, x0T, y = _tc_level0(g0, t0T, kpts[0], w0, bb0,
                               rb_b1["wd"], rb_b1["bd"], b0)
    gy = _gather_rows(y, idx0f)
    x1T, y = _tc_resblock(gy, infl0, x0T, rb_b1["wf"], rb_b1["bk"],
                          rb_b1["wu"], rb_b1["bu"], rb_b1["wsc"],
                          (rb_a1["wd"], rb_a1["bd"]), None, r0)
    gy = _gather_rows(y, idx0f)
    (x2,) = _tc_resblock(gy, infl0, x1T, rb_a1["wf"], rb_a1["bk"],
                         rb_a1["wu"], rb_a1["bu"], rb_a1["wsc"],
                         None, None, r0, out_rm=True)
    skip1 = x2[:n0]

    infl1 = _tc_infl(gpos1, p1T, kpts[1], RADII[1], b1)
    gp1 = _gather_rows(x2, pool1f)
    xT, y = _tc_pooldown(gp1, rbs2[0]["wd"], rbs2[0]["bd"], b1, y_rm=True)
    for ri, rb in enumerate(rbs2):
        gy = _gather_rows(y, idx1f)
        final = ri == len(rbs2) - 1
        nxt = None if final else (rbs2[ri + 1]["wd"], rbs2[ri + 1]["bd"])
        res = _tc_resblock(gy, infl1, xT, rb["wf"], rb["bk"], rb["wu"],
                           rb["bu"], rb["wsc"], nxt, None, r1,
                           out_rm=final)
        if final:
            x_l1 = res[0]
        else:
            xT, y = res
    skip2 = x_l1[:n1]

    infl2 = _tc_infl_oh(p2T, idx2T, kpts[2], RADII[2], b2)
    gp2 = _gather_rows(x_l1, pool2f)
    xT, yT = _tc_pooldown(gp2, rbs3[0]["wd"], rbs3[0]["bd"], b2, y_rm=False)
    for ri, rb in enumerate(rbs3):
        final = ri == len(rbs3) - 1
        nxt = None if final else (rbs3[ri + 1]["wd"], rbs3[ri + 1]["bd"])
        res = _tc_resblock_oh(yT, idx2T, infl2, xT, rb["wf"], rb["bk"],
                              rb["wu"], rb["bu"], rb["wsc"], nxt, None,
                              r2, out_rm=final)
        if final:
            xT_l2, x_l2 = res
        else:
            xT, yT = res
    skip3 = x_l2[:n2]

    infl3 = _tc_infl_oh(p3T, idx3T, kpts[3], RADII[3], b3)
    xT, yT = _tc_pooldown_oh(xT_l2, pool3T, rbs4[0]["wd"], rbs4[0]["bd"], b3)
    xT, yT = _tc_resblock_oh(yT, idx3T, infl3, xT, rbs4[0]["wf"],
                             rbs4[0]["bk"], rbs4[0]["wu"], rbs4[0]["bu"],
                             rbs4[0]["wsc"],
                             (rbs4[1]["wd"], rbs4[1]["bd"]), None, r3)
    (out,) = _tc_resblock_oh(yT, idx3T, infl3, xT, rbs4[1]["wf"],
                             rbs4[1]["bk"], rbs4[1]["wu"], rbs4[1]["bu"],
                             rbs4[1]["wsc"], None, head, r3)

    return (out[:n3], skip1, skip2, skip3)


# speedup vs baseline: 4.0386x; 1.0142x over previous
import functools

import jax
import jax.numpy as jnp
from jax import lax
from jax.experimental import pallas as pl
from jax.experimental.pallas import tpu as pltpu
from jax.experimental.pallas import tpu_sc as plsc

KP = 15
KN = 16
NW = 32
RADII = (0.0625, 0.125, 0.25, 0.5)
F32 = jnp.float32


def _leaky(x):
    return jnp.where(x >= 0.0, x, 0.1 * x)


def _pad_rows(x, n):
    if x.shape[0] == n:
        return x
    return jnp.pad(x, ((0, n - x.shape[0]),) + ((0, 0),) * (x.ndim - 1))


def _sc_plan(B, D):
    bpw = B // NW
    ch = min(128, bpw)
    while bpw % ch:
        ch -= 8
    g = max(1, min(32768 // (ch * D), 8))
    while bpw % (ch * g):
        g -= 1
    sch = ch * g
    return bpw, ch, g, sch, bpw // sch


def _sc_pipeline(table_hbm, idx_hbm, out_hbm, idx_v, buf_a, buf_b,
                 gs_a, gs_b, ws_a, ws_b, base, plan):
    bpw, ch, g, sch, nsch = plan
    pltpu.sync_copy(idx_hbm.at[pl.ds(base, bpw)], idx_v)

    def fire(s, buf, gsem):
        for j in range(g):
            pltpu.async_copy(
                table_hbm.at[idx_v.at[pl.ds(s * sch + j * ch, ch)]],
                buf.at[pl.ds(j * ch, ch)], gsem)

    def retire(s, buf, gsem, wsem):
        pltpu.make_async_copy(table_hbm.at[pl.ds(0, sch)], buf, gsem).wait()
        pltpu.async_copy(buf, out_hbm.at[pl.ds(base + s * sch, sch)],
                         wsem).wait()

    fire(0, buf_a, gs_a)
    if nsch == 1:
        retire(0, buf_a, gs_a, ws_a)
    else:
        @pl.loop(0, nsch // 2)
        def _(h):
            s0 = 2 * h
            fire(s0 + 1, buf_b, gs_b)
            retire(s0, buf_a, gs_a, ws_a)

            @pl.when(s0 + 2 < nsch)
            def _():
                fire(s0 + 2, buf_a, gs_a)
            retire(s0 + 1, buf_b, gs_b, ws_b)
        if nsch % 2:
            retire(nsch - 1, buf_a, gs_a, ws_a)


def _sc_gather_multi(pairs):
    plans = [_sc_plan(idx.shape[0], t.shape[1]) for t, idx in pairs]
    mesh = plsc.VectorSubcoreMesh(core_axis_name="c", subcore_axis_name="s")
    scratch = []
    for (t, idx), (bpw, ch, g, sch, nsch) in zip(pairs, plans):
        scratch += [pltpu.VMEM((bpw,), jnp.int32),
                    pltpu.VMEM((sch, t.shape[1]), F32),
                    pltpu.VMEM((sch, t.shape[1]), F32)]
    scratch += [pltpu.SemaphoreType.DMA] * 4

    @functools.partial(
        pl.kernel, mesh=mesh,
        out_type=[jax.ShapeDtypeStruct((idx.shape[0], t.shape[1]), F32)
                  for t, idx in pairs],
        compiler_params=pltpu.CompilerParams(use_tc_tiling_on_sc=False),
        scratch_types=scratch)
    def gk(*refs):
        n = len(pairs)
        tables = refs[:2 * n:2]
        idxs = refs[1:2 * n:2]
        outs = refs[2 * n:3 * n]
        rest = refs[3 * n:]
        sems = rest[-4:]
        wid = lax.axis_index("s") * 2 + lax.axis_index("c")
        for i, plan in enumerate(plans):
            _sc_pipeline(tables[i], idxs[i], outs[i],
                         rest[3 * i], rest[3 * i + 1], rest[3 * i + 2],
                         sems[0], sems[1], sems[2], sems[3],
                         wid * plan[0], plan)

    flat = []
    for t, idx in pairs:
        flat += [t, idx]
    res = gk(*flat)
    return res if isinstance(res, (list, tuple)) else [res]


def _sc_gather(table, idx_flat):
    _, D = table.shape
    B = idx_flat.shape[0]
    bpw = B // NW
    ch = min(128, bpw)
    while bpw % ch:
        ch -= 8
    g = max(1, min(32768 // (ch * D), 8))
    while bpw % (ch * g):
        g -= 1
    sch = ch * g
    nsch = bpw // sch
    mesh = plsc.VectorSubcoreMesh(core_axis_name="c", subcore_axis_name="s")

    @functools.partial(
        pl.kernel, mesh=mesh,
        out_type=jax.ShapeDtypeStruct((B, D), F32),
        compiler_params=pltpu.CompilerParams(use_tc_tiling_on_sc=False),
        scratch_types=[pltpu.VMEM((bpw,), jnp.int32),
                       pltpu.VMEM((sch, D), F32),
                       pltpu.VMEM((sch, D), F32),
                       pltpu.SemaphoreType.DMA,
                       pltpu.SemaphoreType.DMA,
                       pltpu.SemaphoreType.DMA,
                       pltpu.SemaphoreType.DMA])
    def gk(table_hbm, idx_hbm, out_hbm, idx_v, buf_a, buf_b, gs_a, gs_b,
           ws_a, ws_b):
        wid = lax.axis_index("s") * 2 + lax.axis_index("c")
        base = wid * bpw
        pltpu.sync_copy(idx_hbm.at[pl.ds(base, bpw)], idx_v)

        def fire(s, buf, gsem):
            for j in range(g):
                pltpu.async_copy(
                    table_hbm.at[idx_v.at[pl.ds(s * sch + j * ch, ch)]],
                    buf.at[pl.ds(j * ch, ch)], gsem)

        def retire(s, buf, gsem, wsem):
            pltpu.make_async_copy(table_hbm.at[pl.ds(0, sch)], buf, gsem).wait()
            pltpu.async_copy(buf, out_hbm.at[pl.ds(base + s * sch, sch)],
                             wsem).wait()

        fire(0, buf_a, gs_a)
        if nsch == 1:
            retire(0, buf_a, gs_a, ws_a)
        else:
            @pl.loop(0, nsch // 2)
            def _(h):
                s0 = 2 * h
                fire(s0 + 1, buf_b, gs_b)
                retire(s0, buf_a, gs_a, ws_a)

                @pl.when(s0 + 2 < nsch)
                def _():
                    fire(s0 + 2, buf_a, gs_a)
                retire(s0 + 1, buf_b, gs_b, ws_b)
            if nsch % 2:
                retire(nsch - 1, buf_a, gs_a, ws_a)

    return gk(table, idx_flat)


def _gather_rows(table, idx_flat):
    out = _sc_gather(table, idx_flat)
    return out.reshape(idx_flat.shape[0] // KN, KN * table.shape[1])


def _dot(a, b):
    return jax.lax.dot_general(a, b, (((1,), (0,)), ((), ())),
                               preferred_element_type=F32)


def _kp_cols(kpts):
    return tuple(kpts[:, i].reshape(KP, 1) for i in range(3))


def _infl_rows_t(gt, pxt, pyt, pzt, kxc, kyc, kzc, inv_r, stride=16):
    rows = []
    for k in range(KN):
        o = stride * k
        rx = gt[o:o + 1, :] - pxt
        ry = gt[o + 1:o + 2, :] - pyt
        rz = gt[o + 2:o + 3, :] - pzt
        dx = rx - kxc
        dy = ry - kyc
        dz = rz - kzc
        d = jnp.sqrt(dx * dx + dy * dy + dz * dz + 1e-12)
        rows.append(jnp.maximum(0.0, 1.0 - d * inv_r))
    return rows


def _onehot(idxk, v):
    iota = lax.broadcasted_iota(jnp.int32, (v, 1), 0)
    return jnp.where(idxk == iota, 1.0, 0.0).astype(F32)


def _wspec(shape):
    return pl.BlockSpec(shape, lambda i: tuple(0 for _ in shape))


def _cspec(shape):
    return pl.BlockSpec(shape, lambda i: (0, i))


def _rspec(shape):
    return pl.BlockSpec(shape, lambda i: (i, 0))


def _tc_level0(g0, tT, kpts, w0, b0, wd, bd, blk):
    n = tT.shape[1]
    inv_r = 1.0 / RADII[0]
    c0, m = w0.shape[1], wd.shape[1]
    kxc, kyc, kzc = _kp_cols(kpts)

    def body(g_ref, t_ref, kx_ref, ky_ref, kz_ref, w0_ref, b0_ref,
             wd_ref, bd_ref, infl_ref, x0_ref, y1_ref):
        gt = jnp.transpose(g_ref[...])
        t = t_ref[...]
        rows = _infl_rows_t(gt, t[0:1, :], t[1:2, :], t[2:3, :],
                            kx_ref[...], ky_ref[...], kz_ref[...], inv_r)
        agg = None
        for k in range(KN):
            a = rows[k] * gt[16 * k + 3:16 * k + 4, :]
            agg = a if agg is None else agg + a
        infl_ref[...] = jnp.concatenate(rows, axis=0)
        x0 = _leaky(_dot(w0_ref[...], agg) + b0_ref[...])
        x0_ref[...] = x0
        y1_ref[...] = jnp.transpose(
            _leaky(_dot(wd_ref[...], x0) + bd_ref[...]))

    return pl.pallas_call(
        body,
        grid=(n // blk,),
        in_specs=[_rspec((blk, KN * 16)), _cspec((16, blk)),
                  _wspec((KP, 1)), _wspec((KP, 1)), _wspec((KP, 1)),
                  _wspec((c0, KP)), _wspec((c0, 1)),
                  _wspec((m, c0)), _wspec((m, 1))],
        out_specs=[_cspec((KN * KP, blk)), _cspec((c0, blk)),
                   _rspec((blk, m))],
        out_shape=[jax.ShapeDtypeStruct((KN * KP, n), F32),
                   jax.ShapeDtypeStruct((c0, n), F32),
                   jax.ShapeDtypeStruct((n, m), F32)],
    )(g0, tT, kxc, kyc, kzc, w0.T, b0.T, wd.T, bd.T)


def _tc_infl(gpos, pT, kpts, radius, blk):
    n = pT.shape[1]
    inv_r = 1.0 / radius
    kxc, kyc, kzc = _kp_cols(kpts)

    def body(g_ref, t_ref, kx_ref, ky_ref, kz_ref, infl_ref):
        gt = jnp.transpose(g_ref[...])
        t = t_ref[...]
        rows = _infl_rows_t(gt, t[0:1, :], t[1:2, :], t[2:3, :],
                            kx_ref[...], ky_ref[...], kz_ref[...], inv_r)
        infl_ref[...] = jnp.concatenate(rows, axis=0)

    return pl.pallas_call(
        body,
        grid=(n // blk,),
        in_specs=[_rspec((blk, KN * 16)), _cspec((16, blk)),
                  _wspec((KP, 1)), _wspec((KP, 1)), _wspec((KP, 1))],
        out_specs=_cspec((KN * KP, blk)),
        out_shape=jax.ShapeDtypeStruct((KN * KP, n), F32),
    )(gpos, pT, kxc, kyc, kzc)


def _tc_infl_oh(pT, idxT, kpts, radius, blk):
    v = pT.shape[1]
    n = idxT.shape[1]
    inv_r = 1.0 / radius
    kxc, kyc, kzc = _kp_cols(kpts)

    def body(p_ref, pc_ref, i_ref, kx_ref, ky_ref, kz_ref, infl_ref):
        pt = p_ref[...][0:8, :]
        pc = pc_ref[...]
        idx = i_ref[...]
        pxt, pyt, pzt = pc[0:1, :], pc[1:2, :], pc[2:3, :]
        kxcv, kycv, kzcv = kx_ref[...], ky_ref[...], kz_ref[...]
        rows = []
        for k in range(KN):
            gk = _dot(pt, _onehot(idx[k:k + 1, :], v))
            rx = gk[0:1, :] - pxt
            ry = gk[1:2, :] - pyt
            rz = gk[2:3, :] - pzt
            dx = rx - kxcv
            dy = ry - kycv
            dz = rz - kzcv
            d = jnp.sqrt(dx * dx + dy * dy + dz * dz + 1e-12)
            rows.append(jnp.maximum(0.0, 1.0 - d * inv_r))
        infl_ref[...] = jnp.concatenate(rows, axis=0)

    return pl.pallas_call(
        body,
        grid=(n // blk,),
        in_specs=[_wspec((16, v)), _cspec((16, blk)), _cspec((KN, blk)),
                  _wspec((KP, 1)), _wspec((KP, 1)), _wspec((KP, 1))],
        out_specs=_cspec((KN * KP, blk)),
        out_shape=jax.ShapeDtypeStruct((KN * KP, n), F32),
    )(pT, pT, idxT, kxc, kyc, kzc)


def _tc_pooldown(gp, wd, bd, blk, y_rm):
    n = gp.shape[0]
    cp, m = wd.shape

    def body(g_ref, wd_ref, bd_ref, xp_ref, y_ref):
        g = g_ref[...]
        xp = g[:, :cp]
        for k in range(1, KN):
            xp = jnp.maximum(xp, g[:, k * cp:(k + 1) * cp])
        xpt = jnp.transpose(xp)
        xp_ref[...] = xpt
        yt = _leaky(_dot(wd_ref[...], xpt) + bd_ref[...])
        y_ref[...] = jnp.transpose(yt) if y_rm else yt

    y_spec = _rspec((blk, m)) if y_rm else _cspec((m, blk))
    y_shape = (n, m) if y_rm else (m, n)
    return pl.pallas_call(
        body,
        grid=(n // blk,),
        in_specs=[_rspec((blk, KN * cp)), _wspec((m, cp)), _wspec((m, 1))],
        out_specs=[_cspec((cp, blk)), y_spec],
        out_shape=[jax.ShapeDtypeStruct((cp, n), F32),
                   jax.ShapeDtypeStruct(y_shape, F32)],
    )(gp, wd.T, bd.T)


def _tc_pooldown_oh(xT, poolT, wd, bd, blk):
    cp, v = xT.shape
    n = poolT.shape[1]
    m = wd.shape[1]

    def body(x_ref, i_ref, wd_ref, bd_ref, xp_ref, y_ref):
        xt = x_ref[...]
        idx = i_ref[...]
        xp = None
        for k in range(KN):
            gk = _dot(xt, _onehot(idx[k:k + 1, :], v))
            xp = gk if xp is None else jnp.maximum(xp, gk)
        xp_ref[...] = xp
        y_ref[...] = _leaky(_dot(wd_ref[...], xp) + bd_ref[...])

    return pl.pallas_call(
        body,
        grid=(n // blk,),
        in_specs=[_wspec((cp, v)), _cspec((KN, blk)),
                  _wspec((m, cp)), _wspec((m, 1))],
        out_specs=[_cspec((cp, blk)), _cspec((m, blk))],
        out_shape=[jax.ShapeDtypeStruct((cp, n), F32),
                   jax.ShapeDtypeStruct((m, n), F32)],
    )(xT, poolT, wd.T, bd.T)


def _agg_tail(gk_of_k, iv, wf_ref, bk_ref, wu_ref, bu_ref, wsc_ref, xt):
    accs = [None] * KP
    for k in range(KN):
        gk = gk_of_k(k)
        for p in range(KP):
            w = iv[k * KP + p:k * KP + p + 1, :]
            t = w * gk
            accs[p] = t if accs[p] is None else accs[p] + t
    s = jnp.concatenate(accs, axis=0)
    z = _leaky(_dot(wf_ref[...], s) + bk_ref[...])
    o = _dot(wu_ref[...], z) + bu_ref[...]
    sc = _dot(wsc_ref[...], xt) if wsc_ref is not None else xt
    return _leaky(o + sc)


def _tc_resblock(gy, inflT, xT_in, wf, bk, wu, bu, wsc, dn, head, blk,
                 out_rm=False):
    n = gy.shape[0]
    m = wf.shape[1]
    ci = xT_in.shape[0]
    co = wu.shape[1]
    has_sc = wsc is not None
    wdn, bdn = dn if dn is not None else (None, None)
    wh, bh = head if head is not None else (None, None)

    def body(*refs):
        it = iter(refs)
        g_ref, infl_ref, x_ref, wf_ref, bk_ref, wu_ref, bu_ref = (
            next(it) for _ in range(7))
        wsc_ref = next(it) if has_sc else None
        wdn_ref, bdn_ref = (next(it), next(it)) if dn is not None else (0, 0)
        wh_ref, bh_ref = (next(it), next(it)) if head is not None else (0, 0)
        outs = list(it)

        gt = jnp.transpose(g_ref[...])
        xo = _agg_tail(lambda k: gt[k * m:(k + 1) * m, :], infl_ref[...],
                       wf_ref, bk_ref, wu_ref, bu_ref, wsc_ref, x_ref[...])
        if head is not None:
            outs[0][...] = jnp.transpose(_dot(wh_ref[...], xo) + bh_ref[...])
            return
        outs[0][...] = jnp.transpose(xo) if out_rm else xo
        if dn is not None:
            outs[1][...] = jnp.transpose(
                _leaky(_dot(wdn_ref[...], xo) + bdn_ref[...]))

    in_specs = [_rspec((blk, KN * m)), _cspec((KN * KP, blk)),
                _cspec((ci, blk)),
                _wspec((m, KP * m)), _wspec((m, 1)),
                _wspec((co, m)), _wspec((co, 1))]
    args = [gy, inflT, xT_in, wf.T, bk.T, wu.T, bu.T]
    if has_sc:
        in_specs.append(_wspec((co, ci)))
        args.append(wsc.T)
    if dn is not None:
        mn = wdn.shape[1]
        in_specs += [_wspec((mn, co)), _wspec((mn, 1))]
        args += [wdn.T, bdn.T]
    if head is not None:
        chh = wh.shape[1]
        in_specs += [_wspec((chh, co)), _wspec((chh, 1))]
        args += [wh.T, bh.T]

    if head is not None:
        out_specs = [_rspec((blk, wh.shape[1]))]
        out_shape = [jax.ShapeDtypeStruct((n, wh.shape[1]), F32)]
    else:
        out_specs = [_rspec((blk, co)) if out_rm else _cspec((co, blk))]
        out_shape = [jax.ShapeDtypeStruct((n, co) if out_rm else (co, n), F32)]
        if dn is not None:
            mn = wdn.shape[1]
            out_specs.append(_rspec((blk, mn)))
            out_shape.append(jax.ShapeDtypeStruct((n, mn), F32))

    return pl.pallas_call(body, grid=(n // blk,), in_specs=in_specs,
                          out_specs=out_specs, out_shape=out_shape)(*args)


def _tc_resblock_oh(yT, idxT, inflT, xT_in, wf, bk, wu, bu, wsc, dn, head,
                    blk, out_rm=False):
    m, v = yT.shape
    n = idxT.shape[1]
    ci = xT_in.shape[0]
    co = wu.shape[1]
    has_sc = wsc is not None
    wdn, bdn = dn if dn is not None else (None, None)
    wh, bh = head if head is not None else (None, None)

    def body(*refs):
        it = iter(refs)
        y_ref, i_ref, infl_ref, x_ref, wf_ref, bk_ref, wu_ref, bu_ref = (
            next(it) for _ in range(8))
        wsc_ref = next(it) if has_sc else None
        wdn_ref, bdn_ref = (next(it), next(it)) if dn is not None else (0, 0)
        wh_ref, bh_ref = (next(it), next(it)) if head is not None else (0, 0)
        outs = list(it)

        yt = y_ref[...]
        idx = i_ref[...]
        xo = _agg_tail(lambda k: _dot(yt, _onehot(idx[k:k + 1, :], v)),
                       infl_ref[...], wf_ref, bk_ref, wu_ref, bu_ref,
                       wsc_ref, x_ref[...])
        if head is not None:
            outs[0][...] = jnp.transpose(_dot(wh_ref[...], xo) + bh_ref[...])
            return
        outs[0][...] = xo
        i = 1
        if out_rm:
            outs[i][...] = jnp.transpose(xo)
            i += 1
        if dn is not None:
            outs[i][...] = _leaky(_dot(wdn_ref[...], xo) + bdn_ref[...])

    in_specs = [_wspec((m, v)), _cspec((KN, blk)), _cspec((KN * KP, blk)),
                _cspec((ci, blk)),
                _wspec((m, KP * m)), _wspec((m, 1)),
                _wspec((co, m)), _wspec((co, 1))]
    args = [yT, idxT, inflT, xT_in, wf.T, bk.T, wu.T, bu.T]
    if has_sc:
        in_specs.append(_wspec((co, ci)))
        args.append(wsc.T)
    if dn is not None:
        mn = wdn.shape[1]
        in_specs += [_wspec((mn, co)), _wspec((mn, 1))]
        args += [wdn.T, bdn.T]
    if head is not None:
        chh = wh.shape[1]
        in_specs += [_wspec((chh, co)), _wspec((chh, 1))]
        args += [wh.T, bh.T]

    if head is not None:
        out_specs = [_rspec((blk, wh.shape[1]))]
        out_shape = [jax.ShapeDtypeStruct((n, wh.shape[1]), F32)]
    else:
        out_specs = [_cspec((co, blk))]
        out_shape = [jax.ShapeDtypeStruct((co, n), F32)]
        if out_rm:
            out_specs.append(_rspec((blk, co)))
            out_shape.append(jax.ShapeDtypeStruct((n, co), F32))
        if dn is not None:
            mn = wdn.shape[1]
            out_specs.append(_cspec((mn, blk)))
            out_shape.append(jax.ShapeDtypeStruct((mn, n), F32))

    return pl.pallas_call(body, grid=(n // blk,), in_specs=in_specs,
                          out_specs=out_specs, out_shape=out_shape)(*args)


def _fold_lin(w, bn):
    return w * bn["g"][None, :], bn["b"].reshape(1, -1)


def _fold_kp(kpw, bn):
    pm, m = kpw.shape[0] * kpw.shape[1], kpw.shape[2]
    return kpw.reshape(pm, m) * bn["g"][None, :], bn["b"].reshape(1, -1)


def _rb_weights(p):
    wd, bd = _fold_lin(p["down"], p["bnd"])
    wf, bk = _fold_kp(p["kp"], p["bnk"])
    wu, bu = _fold_lin(p["up"], p["bnu"])
    return dict(wd=wd, bd=bd, wf=wf, bk=bk, wu=wu, bu=bu, wsc=p.get("sc"))


def kernel(points, features, idx0, idx1, idx2, idx3, pool1, pool2, pool3,
           pos1, pos2, pos3, params):
    n0, n1, n2, n3 = points.shape[0], pos1.shape[0], pos2.shape[0], pos3.shape[0]
    np0, np1, np2, np3 = 10240, 2560, 640, 160
    b0, b1, b2, b3 = 256, 256, 128, 160
    r0, r1, r2, r3 = 1024, 512, 128, 160

    def padw(x, w=16):
        return jnp.pad(x, ((0, 0), (0, w - x.shape[1])))

    t0 = _pad_rows(padw(jnp.concatenate([points, features], axis=1)), np0)
    p1 = _pad_rows(padw(pos1), np1)
    p2 = _pad_rows(padw(pos2), np2)
    p3 = _pad_rows(padw(pos3), np3)
    t0T, p1T, p2T, p3T = t0.T, p1.T, p2.T, p3.T
    idx0f = _pad_rows(idx0, np0).reshape(-1)
    idx1f = _pad_rows(idx1, np1).reshape(-1)
    idx2T = _pad_rows(idx2, np2).T
    idx3T = idx3.T
    pool1f = _pad_rows(pool1, np1).reshape(-1)
    pool2f = _pad_rows(pool2, np2).reshape(-1)
    pool3T = pool3.T

    kpts = params["kpts"]
    w0, bb0 = _fold_kp(params["b1_kp"].reshape(KP, 1, 64), params["b1_bn"])
    rb_b1 = _rb_weights(params["b1_rb"])
    rb_a1 = _rb_weights(params["b1_ra"])
    rbs2 = [_rb_weights(p) for p in params["b2"]]
    rbs3 = [_rb_weights(p) for p in params["b3"]]
    rbs4 = [_rb_weights(p) for p in params["b4"]]
    head = (params["head_w"], params["head_b"].reshape(1, -1))

    g0r, gpos1r = _sc_gather_multi([(t0, idx0f), (p1, idx1f)])
    g0 = g0r.reshape(np0, KN * 16)
    gpos1 = gpos1r.reshape(np1, KN * 16)
    infl0, x0T, y = _tc_level0(g0, t0T, kpts[0], w0, bb0,
                               rb_b1["wd"], rb_b1["bd"], b0)
    gy = _gather_rows(y, idx0f)
    x1T, y = _tc_resblock(gy, infl0, x0T, rb_b1["wf"], rb_b1["bk"],
                          rb_b1["wu"], rb_b1["bu"], rb_b1["wsc"],
                          (rb_a1["wd"], rb_a1["bd"]), None, r0)
    gy = _gather_rows(y, idx0f)
    (x2,) = _tc_resblock(gy, infl0, x1T, rb_a1["wf"], rb_a1["bk"],
                         rb_a1["wu"], rb_a1["bu"], rb_a1["wsc"],
                         None, None, r0, out_rm=True)
    skip1 = x2[:n0]

    infl1 = _tc_infl(gpos1, p1T, kpts[1], RADII[1], b1)
    gp1 = _gather_rows(x2, pool1f)
    xT, y = _tc_pooldown(gp1, rbs2[0]["wd"], rbs2[0]["bd"], b1, y_rm=True)
    for ri, rb in enumerate(rbs2):
        gy = _gather_rows(y, idx1f)
        final = ri == len(rbs2) - 1
        nxt = None if final else (rbs2[ri + 1]["wd"], rbs2[ri + 1]["bd"])
        res = _tc_resblock(gy, infl1, xT, rb["wf"], rb["bk"], rb["wu"],
                           rb["bu"], rb["wsc"], nxt, None, r1,
                           out_rm=final)
        if final:
            x_l1 = res[0]
        else:
            xT, y = res
    skip2 = x_l1[:n1]

    infl2 = _tc_infl_oh(p2T, idx2T, kpts[2], RADII[2], b2)
    gp2 = _gather_rows(x_l1, pool2f)
    xT, yT = _tc_pooldown(gp2, rbs3[0]["wd"], rbs3[0]["bd"], b2, y_rm=False)
    for ri, rb in enumerate(rbs3):
        final = ri == len(rbs3) - 1
        nxt = None if final else (rbs3[ri + 1]["wd"], rbs3[ri + 1]["bd"])
        res = _tc_resblock_oh(yT, idx2T, infl2, xT, rb["wf"], rb["bk"],
                              rb["wu"], rb["bu"], rb["wsc"], nxt, None,
                              r2, out_rm=final)
        if final:
            xT_l2, x_l2 = res
        else:
            xT, yT = res
    skip3 = x_l2[:n2]

    infl3 = _tc_infl_oh(p3T, idx3T, kpts[3], RADII[3], b3)
    xT, yT = _tc_pooldown_oh(xT_l2, pool3T, rbs4[0]["wd"], rbs4[0]["bd"], b3)
    xT, yT = _tc_resblock_oh(yT, idx3T, infl3, xT, rbs4[0]["wf"],
                             rbs4[0]["bk"], rbs4[0]["wu"], rbs4[0]["bu"],
                             rbs4[0]["wsc"],
                             (rbs4[1]["wd"], rbs4[1]["bd"]), None, r3)
    (out,) = _tc_resblock_oh(yT, idx3T, infl3, xT, rbs4[1]["wf"],
                             rbs4[1]["bk"], rbs4[1]["wu"], rbs4[1]["bu"],
                             rbs4[1]["wsc"], None, head, r3)

    return (out[:n3], skip1, skip2, skip3)


# speedup vs baseline: 4.0706x; 1.0079x over previous
import functools

import jax
import jax.numpy as jnp
from jax import lax
from jax.experimental import pallas as pl
from jax.experimental.pallas import tpu as pltpu
from jax.experimental.pallas import tpu_sc as plsc

KP = 15
KN = 16
NW = 32
RADII = (0.0625, 0.125, 0.25, 0.5)
F32 = jnp.float32
BF16 = jnp.bfloat16


def _leaky(x):
    return jnp.where(x >= 0.0, x, 0.1 * x)


def _pad_rows(x, n):
    if x.shape[0] == n:
        return x
    return jnp.pad(x, ((0, n - x.shape[0]),) + ((0, 0),) * (x.ndim - 1))


def _sc_plan(B, D, esize=4):
    bpw = B // NW
    ch = min(128, bpw)
    while bpw % ch:
        ch -= 8
    g = max(1, min(131072 // (ch * D * esize), 8))
    while bpw % (ch * g):
        g -= 1
    sch = ch * g
    return bpw, ch, g, sch, bpw // sch


def _sc_pipeline(table_hbm, idx_hbm, out_hbm, idx_v, buf_a, buf_b,
                 gs_a, gs_b, ws_a, ws_b, base, plan):
    bpw, ch, g, sch, nsch = plan
    pltpu.sync_copy(idx_hbm.at[pl.ds(base, bpw)], idx_v)

    def fire(s, buf, gsem):
        for j in range(g):
            pltpu.async_copy(
                table_hbm.at[idx_v.at[pl.ds(s * sch + j * ch, ch)]],
                buf.at[pl.ds(j * ch, ch)], gsem)

    def retire(s, buf, gsem, wsem):
        pltpu.make_async_copy(table_hbm.at[pl.ds(0, sch)], buf, gsem).wait()
        pltpu.async_copy(buf, out_hbm.at[pl.ds(base + s * sch, sch)],
                         wsem).wait()

    fire(0, buf_a, gs_a)
    if nsch == 1:
        retire(0, buf_a, gs_a, ws_a)
    else:
        @pl.loop(0, nsch // 2)
        def _(h):
            s0 = 2 * h
            fire(s0 + 1, buf_b, gs_b)
            retire(s0, buf_a, gs_a, ws_a)

            @pl.when(s0 + 2 < nsch)
            def _():
                fire(s0 + 2, buf_a, gs_a)
            retire(s0 + 1, buf_b, gs_b, ws_b)
        if nsch % 2:
            retire(nsch - 1, buf_a, gs_a, ws_a)


def _sc_gather_multi(pairs):
    plans = [_sc_plan(idx.shape[0], t.shape[1], t.dtype.itemsize)
             for t, idx in pairs]
    mesh = plsc.VectorSubcoreMesh(core_axis_name="c", subcore_axis_name="s")
    scratch = []
    for (t, idx), (bpw, ch, g, sch, nsch) in zip(pairs, plans):
        scratch += [pltpu.VMEM((bpw,), jnp.int32),
                    pltpu.VMEM((sch, t.shape[1]), t.dtype),
                    pltpu.VMEM((sch, t.shape[1]), t.dtype)]
    scratch += [pltpu.SemaphoreType.DMA] * 4

    @functools.partial(
        pl.kernel, mesh=mesh,
        out_type=[jax.ShapeDtypeStruct((idx.shape[0], t.shape[1]), t.dtype)
                  for t, idx in pairs],
        compiler_params=pltpu.CompilerParams(use_tc_tiling_on_sc=False),
        scratch_types=scratch)
    def gk(*refs):
        n = len(pairs)
        tables = refs[:2 * n:2]
        idxs = refs[1:2 * n:2]
        outs = refs[2 * n:3 * n]
        rest = refs[3 * n:]
        sems = rest[-4:]
        wid = lax.axis_index("s") * 2 + lax.axis_index("c")
        for i, plan in enumerate(plans):
            _sc_pipeline(tables[i], idxs[i], outs[i],
                         rest[3 * i], rest[3 * i + 1], rest[3 * i + 2],
                         sems[0], sems[1], sems[2], sems[3],
                         wid * plan[0], plan)

    flat = []
    for t, idx in pairs:
        flat += [t, idx]
    res = gk(*flat)
    return res if isinstance(res, (list, tuple)) else [res]


def _sc_gather(table, idx_flat):
    _, D = table.shape
    B = idx_flat.shape[0]
    dt = table.dtype
    bpw, ch, g, sch, nsch = _sc_plan(B, D, table.dtype.itemsize)
    mesh = plsc.VectorSubcoreMesh(core_axis_name="c", subcore_axis_name="s")

    @functools.partial(
        pl.kernel, mesh=mesh,
        out_type=jax.ShapeDtypeStruct((B, D), dt),
        compiler_params=pltpu.CompilerParams(use_tc_tiling_on_sc=False),
        scratch_types=[pltpu.VMEM((bpw,), jnp.int32),
                       pltpu.VMEM((sch, D), dt),
                       pltpu.VMEM((sch, D), dt),
                       pltpu.SemaphoreType.DMA,
                       pltpu.SemaphoreType.DMA,
                       pltpu.SemaphoreType.DMA,
                       pltpu.SemaphoreType.DMA])
    def gk(table_hbm, idx_hbm, out_hbm, idx_v, buf_a, buf_b, gs_a, gs_b,
           ws_a, ws_b):
        wid = lax.axis_index("s") * 2 + lax.axis_index("c")
        base = wid * bpw
        pltpu.sync_copy(idx_hbm.at[pl.ds(base, bpw)], idx_v)

        def fire(s, buf, gsem):
            for j in range(g):
                pltpu.async_copy(
                    table_hbm.at[idx_v.at[pl.ds(s * sch + j * ch, ch)]],
                    buf.at[pl.ds(j * ch, ch)], gsem)

        def retire(s, buf, gsem, wsem):
            pltpu.make_async_copy(table_hbm.at[pl.ds(0, sch)], buf, gsem).wait()
            pltpu.async_copy(buf, out_hbm.at[pl.ds(base + s * sch, sch)],
                             wsem).wait()

        fire(0, buf_a, gs_a)
        if nsch == 1:
            retire(0, buf_a, gs_a, ws_a)
        else:
            @pl.loop(0, nsch // 2)
            def _(h):
                s0 = 2 * h
                fire(s0 + 1, buf_b, gs_b)
                retire(s0, buf_a, gs_a, ws_a)

                @pl.when(s0 + 2 < nsch)
                def _():
                    fire(s0 + 2, buf_a, gs_a)
                retire(s0 + 1, buf_b, gs_b, ws_b)
            if nsch % 2:
                retire(nsch - 1, buf_a, gs_a, ws_a)

    return gk(table, idx_flat)


def _gather_rows(table, idx_flat):
    out = _sc_gather(table, idx_flat)
    return out.reshape(idx_flat.shape[0] // KN, KN * table.shape[1])


def _dot(a, b):
    return jax.lax.dot_general(a, b, (((1,), (0,)), ((), ())),
                               preferred_element_type=F32)


def _kp_cols(kpts):
    return tuple(kpts[:, i].reshape(KP, 1) for i in range(3))


def _infl_rows_t(gt, pxt, pyt, pzt, kxc, kyc, kzc, inv_r, stride=16):
    rows = []
    for k in range(KN):
        o = stride * k
        rx = gt[o:o + 1, :] - pxt
        ry = gt[o + 1:o + 2, :] - pyt
        rz = gt[o + 2:o + 3, :] - pzt
        dx = rx - kxc
        dy = ry - kyc
        dz = rz - kzc
        d = jnp.sqrt(dx * dx + dy * dy + dz * dz + 1e-12)
        rows.append(jnp.maximum(0.0, 1.0 - d * inv_r))
    return rows


def _onehot(idxk, v):
    iota = lax.broadcasted_iota(jnp.int32, (v, 1), 0)
    return jnp.where(idxk == iota, 1.0, 0.0).astype(F32)


def _wspec(shape):
    return pl.BlockSpec(shape, lambda i: tuple(0 for _ in shape))


def _cspec(shape):
    return pl.BlockSpec(shape, lambda i: (0, i))


def _rspec(shape):
    return pl.BlockSpec(shape, lambda i: (i, 0))


def _tc_level0(g0, tT, kpts, w0, b0, wd, bd, blk):
    n = tT.shape[1]
    inv_r = 1.0 / RADII[0]
    c0, m = w0.shape[1], wd.shape[1]
    kxc, kyc, kzc = _kp_cols(kpts)

    def body(g_ref, t_ref, kx_ref, ky_ref, kz_ref, w0_ref, b0_ref,
             wd_ref, bd_ref, infl_ref, x0_ref, y1_ref):
        gt = jnp.transpose(g_ref[...])
        t = t_ref[...]
        rows = _infl_rows_t(gt, t[0:1, :], t[1:2, :], t[2:3, :],
                            kx_ref[...], ky_ref[...], kz_ref[...], inv_r)
        agg = None
        for k in range(KN):
            a = rows[k] * gt[16 * k + 3:16 * k + 4, :]
            agg = a if agg is None else agg + a
        infl_ref[...] = jnp.concatenate(rows, axis=0)
        x0 = _leaky(_dot(w0_ref[...], agg) + b0_ref[...])
        x0_ref[...] = x0
        y1_ref[...] = jnp.transpose(
            _leaky(_dot(wd_ref[...], x0) + bd_ref[...])).astype(BF16)

    return pl.pallas_call(
        body,
        grid=(n // blk,),
        in_specs=[_rspec((blk, KN * 16)), _cspec((16, blk)),
                  _wspec((KP, 1)), _wspec((KP, 1)), _wspec((KP, 1)),
                  _wspec((c0, KP)), _wspec((c0, 1)),
                  _wspec((m, c0)), _wspec((m, 1))],
        out_specs=[_cspec((KN * KP, blk)), _cspec((c0, blk)),
                   _rspec((blk, m))],
        out_shape=[jax.ShapeDtypeStruct((KN * KP, n), F32),
                   jax.ShapeDtypeStruct((c0, n), F32),
                   jax.ShapeDtypeStruct((n, m), BF16)],
    )(g0, tT, kxc, kyc, kzc, w0.T, b0.T, wd.T, bd.T)


def _tc_infl(gpos, pT, kpts, radius, blk):
    n = pT.shape[1]
    inv_r = 1.0 / radius
    kxc, kyc, kzc = _kp_cols(kpts)

    def body(g_ref, t_ref, kx_ref, ky_ref, kz_ref, infl_ref):
        gt = jnp.transpose(g_ref[...])
        t = t_ref[...]
        rows = _infl_rows_t(gt, t[0:1, :], t[1:2, :], t[2:3, :],
                            kx_ref[...], ky_ref[...], kz_ref[...], inv_r)
        infl_ref[...] = jnp.concatenate(rows, axis=0)

    return pl.pallas_call(
        body,
        grid=(n // blk,),
        in_specs=[_rspec((blk, KN * 16)), _cspec((16, blk)),
                  _wspec((KP, 1)), _wspec((KP, 1)), _wspec((KP, 1))],
        out_specs=_cspec((KN * KP, blk)),
        out_shape=jax.ShapeDtypeStruct((KN * KP, n), F32),
    )(gpos, pT, kxc, kyc, kzc)


def _tc_infl_oh(pT, idxT, kpts, radius, blk):
    v = pT.shape[1]
    n = idxT.shape[1]
    inv_r = 1.0 / radius
    kxc, kyc, kzc = _kp_cols(kpts)

    def body(p_ref, pc_ref, i_ref, kx_ref, ky_ref, kz_ref, infl_ref):
        pt = p_ref[...][0:8, :]
        pc = pc_ref[...]
        idx = i_ref[...]
        pxt, pyt, pzt = pc[0:1, :], pc[1:2, :], pc[2:3, :]
        kxcv, kycv, kzcv = kx_ref[...], ky_ref[...], kz_ref[...]
        rows = []
        for k in range(KN):
            gk = _dot(pt, _onehot(idx[k:k + 1, :], v))
            rx = gk[0:1, :] - pxt
            ry = gk[1:2, :] - pyt
            rz = gk[2:3, :] - pzt
            dx = rx - kxcv
            dy = ry - kycv
            dz = rz - kzcv
            d = jnp.sqrt(dx * dx + dy * dy + dz * dz + 1e-12)
            rows.append(jnp.maximum(0.0, 1.0 - d * inv_r))
        infl_ref[...] = jnp.concatenate(rows, axis=0)

    return pl.pallas_call(
        body,
        grid=(n // blk,),
        in_specs=[_wspec((16, v)), _cspec((16, blk)), _cspec((KN, blk)),
                  _wspec((KP, 1)), _wspec((KP, 1)), _wspec((KP, 1))],
        out_specs=_cspec((KN * KP, blk)),
        out_shape=jax.ShapeDtypeStruct((KN * KP, n), F32),
    )(pT, pT, idxT, kxc, kyc, kzc)


def _tc_pooldown(gp, wd, bd, blk, y_rm):
    n = gp.shape[0]
    cp, m = wd.shape

    def body(g_ref, wd_ref, bd_ref, xp_ref, y_ref):
        g = g_ref[...].astype(F32)
        xp = g[:, :cp]
        for k in range(1, KN):
            xp = jnp.maximum(xp, g[:, k * cp:(k + 1) * cp])
        xpt = jnp.transpose(xp)
        xp_ref[...] = xpt
        yt = _leaky(_dot(wd_ref[...], xpt) + bd_ref[...])
        y_ref[...] = jnp.transpose(yt).astype(BF16) if y_rm else yt

    y_spec = _rspec((blk, m)) if y_rm else _cspec((m, blk))
    y_shape, y_dt = ((n, m), BF16) if y_rm else ((m, n), F32)
    return pl.pallas_call(
        body,
        grid=(n // blk,),
        in_specs=[_rspec((blk, KN * cp)), _wspec((m, cp)), _wspec((m, 1))],
        out_specs=[_cspec((cp, blk)), y_spec],
        out_shape=[jax.ShapeDtypeStruct((cp, n), F32),
                   jax.ShapeDtypeStruct(y_shape, y_dt)],
    )(gp, wd.T, bd.T)


def _tc_pooldown_oh(xT, poolT, wd, bd, blk):
    cp, v = xT.shape
    n = poolT.shape[1]
    m = wd.shape[1]

    def body(x_ref, i_ref, wd_ref, bd_ref, xp_ref, y_ref):
        xt = x_ref[...]
        idx = i_ref[...]
        xp = None
        for k in range(KN):
            gk = _dot(xt, _onehot(idx[k:k + 1, :], v))
            xp = gk if xp is None else jnp.maximum(xp, gk)
        xp_ref[...] = xp
        y_ref[...] = _leaky(_dot(wd_ref[...], xp) + bd_ref[...])

    return pl.pallas_call(
        body,
        grid=(n // blk,),
        in_specs=[_wspec((cp, v)), _cspec((KN, blk)),
                  _wspec((m, cp)), _wspec((m, 1))],
        out_specs=[_cspec((cp, blk)), _cspec((m, blk))],
        out_shape=[jax.ShapeDtypeStruct((cp, n), F32),
                   jax.ShapeDtypeStruct((m, n), F32)],
    )(xT, poolT, wd.T, bd.T)


def _agg_tail(gk_of_k, iv, wf_ref, bk_ref, wu_ref, bu_ref, wsc_ref, xt):
    accs = [None] * KP
    for k in range(KN):
        gk = gk_of_k(k)
        for p in range(KP):
            w = iv[k * KP + p:k * KP + p + 1, :]
            t = w * gk
            accs[p] = t if accs[p] is None else accs[p] + t
    s = jnp.concatenate(accs, axis=0)
    z = _leaky(_dot(wf_ref[...], s) + bk_ref[...])
    o = _dot(wu_ref[...], z) + bu_ref[...]
    sc = _dot(wsc_ref[...], xt) if wsc_ref is not None else xt
    return _leaky(o + sc)


def _tc_resblock(gy, inflT, xT_in, wf, bk, wu, bu, wsc, dn, head, blk,
                 out_rm=False, bf16_twin=False):
    n = gy.shape[0]
    m = wf.shape[1]
    ci = xT_in.shape[0]
    co = wu.shape[1]
    has_sc = wsc is not None
    wdn, bdn = dn if dn is not None else (None, None)
    wh, bh = head if head is not None else (None, None)

    def body(*refs):
        it = iter(refs)
        g_ref, infl_ref, x_ref, wf_ref, bk_ref, wu_ref, bu_ref = (
            next(it) for _ in range(7))
        wsc_ref = next(it) if has_sc else None
        wdn_ref, bdn_ref = (next(it), next(it)) if dn is not None else (0, 0)
        wh_ref, bh_ref = (next(it), next(it)) if head is not None else (0, 0)
        outs = list(it)

        gt = jnp.transpose(g_ref[...].astype(F32))
        xo = _agg_tail(lambda k: gt[k * m:(k + 1) * m, :], infl_ref[...],
                       wf_ref, bk_ref, wu_ref, bu_ref, wsc_ref, x_ref[...])
        if head is not None:
            outs[0][...] = jnp.transpose(_dot(wh_ref[...], xo) + bh_ref[...])
            return
        outs[0][...] = jnp.transpose(xo) if out_rm else xo
        if bf16_twin:
            outs[1][...] = jnp.transpose(xo).astype(BF16)
        if dn is not None:
            outs[1 + bf16_twin][...] = jnp.transpose(
                _leaky(_dot(wdn_ref[...], xo) + bdn_ref[...])).astype(BF16)

    in_specs = [_rspec((blk, KN * m)), _cspec((KN * KP, blk)),
                _cspec((ci, blk)),
                _wspec((m, KP * m)), _wspec((m, 1)),
                _wspec((co, m)), _wspec((co, 1))]
    args = [gy, inflT, xT_in, wf.T, bk.T, wu.T, bu.T]
    if has_sc:
        in_specs.append(_wspec((co, ci)))
        args.append(wsc.T)
    if dn is not None:
        mn = wdn.shape[1]
        in_specs += [_wspec((mn, co)), _wspec((mn, 1))]
        args += [wdn.T, bdn.T]
    if head is not None:
        chh = wh.shape[1]
        in_specs += [_wspec((chh, co)), _wspec((chh, 1))]
        args += [wh.T, bh.T]

    if head is not None:
        out_specs = [_rspec((blk, wh.shape[1]))]
        out_shape = [jax.ShapeDtypeStruct((n, wh.shape[1]), F32)]
    else:
        out_specs = [_rspec((blk, co)) if out_rm else _cspec((co, blk))]
        out_shape = [jax.ShapeDtypeStruct((n, co) if out_rm else (co, n), F32)]
        if bf16_twin:
            out_specs.append(_rspec((blk, co)))
            out_shape.append(jax.ShapeDtypeStruct((n, co), BF16))
        if dn is not None:
            mn = wdn.shape[1]
            out_specs.append(_rspec((blk, mn)))
            out_shape.append(jax.ShapeDtypeStruct((n, mn), BF16))

    return pl.pallas_call(body, grid=(n // blk,), in_specs=in_specs,
                          out_specs=out_specs, out_shape=out_shape)(*args)


def _tc_resblock_oh(yT, idxT, inflT, xT_in, wf, bk, wu, bu, wsc, dn, head,
                    blk, out_rm=False):
    m, v = yT.shape
    n = idxT.shape[1]
    ci = xT_in.shape[0]
    co = wu.shape[1]
    has_sc = wsc is not None
    wdn, bdn = dn if dn is not None else (None, None)
    wh, bh = head if head is not None else (None, None)

    def body(*refs):
        it = iter(refs)
        y_ref, i_ref, infl_ref, x_ref, wf_ref, bk_ref, wu_ref, bu_ref = (
            next(it) for _ in range(8))
        wsc_ref = next(it) if has_sc else None
        wdn_ref, bdn_ref = (next(it), next(it)) if dn is not None else (0, 0)
        wh_ref, bh_ref = (next(it), next(it)) if head is not None else (0, 0)
        outs = list(it)

        yt = y_ref[...]
        idx = i_ref[...]
        xo = _agg_tail(lambda k: _dot(yt, _onehot(idx[k:k + 1, :], v)),
                       infl_ref[...], wf_ref, bk_ref, wu_ref, bu_ref,
                       wsc_ref, x_ref[...])
        if head is not None:
            outs[0][...] = jnp.transpose(_dot(wh_ref[...], xo) + bh_ref[...])
            return
        outs[0][...] = xo
        i = 1
        if out_rm:
            outs[i][...] = jnp.transpose(xo)
            i += 1
        if dn is not None:
            outs[i][...] = _leaky(_dot(wdn_ref[...], xo) + bdn_ref[...])

    in_specs = [_wspec((m, v)), _cspec((KN, blk)), _cspec((KN * KP, blk)),
                _cspec((ci, blk)),
                _wspec((m, KP * m)), _wspec((m, 1)),
                _wspec((co, m)), _wspec((co, 1))]
    args = [yT, idxT, inflT, xT_in, wf.T, bk.T, wu.T, bu.T]
    if has_sc:
        in_specs.append(_wspec((co, ci)))
        args.append(wsc.T)
    if dn is not None:
        mn = wdn.shape[1]
        in_specs += [_wspec((mn, co)), _wspec((mn, 1))]
        args += [wdn.T, bdn.T]
    if head is not None:
        chh = wh.shape[1]
        in_specs += [_wspec((chh, co)), _wspec((chh, 1))]
        args += [wh.T, bh.T]

    if head is not None:
        out_specs = [_rspec((blk, wh.shape[1]))]
        out_shape = [jax.ShapeDtypeStruct((n, wh.shape[1]), F32)]
    else:
        out_specs = [_cspec((co, blk))]
        out_shape = [jax.ShapeDtypeStruct((co, n), F32)]
        if out_rm:
            out_specs.append(_rspec((blk, co)))
            out_shape.append(jax.ShapeDtypeStruct((n, co), F32))
        if dn is not None:
            mn = wdn.shape[1]
            out_specs.append(_cspec((mn, blk)))
            out_shape.append(jax.ShapeDtypeStruct((mn, n), F32))

    return pl.pallas_call(body, grid=(n // blk,), in_specs=in_specs,
                          out_specs=out_specs, out_shape=out_shape)(*args)


def _fold_lin(w, bn):
    return w * bn["g"][None, :], bn["b"].reshape(1, -1)


def _fold_kp(kpw, bn):
    pm, m = kpw.shape[0] * kpw.shape[1], kpw.shape[2]
    return kpw.reshape(pm, m) * bn["g"][None, :], bn["b"].reshape(1, -1)


def _rb_weights(p):
    wd, bd = _fold_lin(p["down"], p["bnd"])
    wf, bk = _fold_kp(p["kp"], p["bnk"])
    wu, bu = _fold_lin(p["up"], p["bnu"])
    return dict(wd=wd, bd=bd, wf=wf, bk=bk, wu=wu, bu=bu, wsc=p.get("sc"))


def kernel(points, features, idx0, idx1, idx2, idx3, pool1, pool2, pool3,
           pos1, pos2, pos3, params):
    n0, n1, n2, n3 = points.shape[0], pos1.shape[0], pos2.shape[0], pos3.shape[0]
    np0, np1, np2, np3 = 10240, 2560, 640, 160
    b0, b1, b2, b3 = 256, 256, 128, 160
    r0, r1, r2, r3 = 1024, 512, 128, 160

    def padw(x, w=16):
        return jnp.pad(x, ((0, 0), (0, w - x.shape[1])))

    t0 = _pad_rows(padw(jnp.concatenate([points, features], axis=1)), np0)
    p1 = _pad_rows(padw(pos1), np1)
    p2 = _pad_rows(padw(pos2), np2)
    p3 = _pad_rows(padw(pos3), np3)
    t0T, p1T, p2T, p3T = t0.T, p1.T, p2.T, p3.T
    idx0f = _pad_rows(idx0, np0).reshape(-1)
    idx1f = _pad_rows(idx1, np1).reshape(-1)
    idx2T = _pad_rows(idx2, np2).T
    idx3T = idx3.T
    pool1f = _pad_rows(pool1, np1).reshape(-1)
    pool2f = _pad_rows(pool2, np2).reshape(-1)
    pool3T = pool3.T

    kpts = params["kpts"]
    w0, bb0 = _fold_kp(params["b1_kp"].reshape(KP, 1, 64), params["b1_bn"])
    rb_b1 = _rb_weights(params["b1_rb"])
    rb_a1 = _rb_weights(params["b1_ra"])
    rbs2 = [_rb_weights(p) for p in params["b2"]]
    rbs3 = [_rb_weights(p) for p in params["b3"]]
    rbs4 = [_rb_weights(p) for p in params["b4"]]
    head = (params["head_w"], params["head_b"].reshape(1, -1))

    g0r, gpos1r = _sc_gather_multi([(t0, idx0f), (p1, idx1f)])
    g0 = g0r.reshape(np0, KN * 16)
    gpos1 = gpos1r.reshape(np1, KN * 16)
    infl0, x0T, y = _tc_level0(g0, t0T, kpts[0], w0, bb0,
                               rb_b1["wd"], rb_b1["bd"], b0)
    gy = _gather_rows(y, idx0f)
    x1T, y = _tc_resblock(gy, infl0, x0T, rb_b1["wf"], rb_b1["bk"],
                          rb_b1["wu"], rb_b1["bu"], rb_b1["wsc"],
                          (rb_a1["wd"], rb_a1["bd"]), None, r0)
    gy = _gather_rows(y, idx0f)
    x2, x2bf = _tc_resblock(gy, infl0, x1T, rb_a1["wf"], rb_a1["bk"],
                            rb_a1["wu"], rb_a1["bu"], rb_a1["wsc"],
                            None, None, r0, out_rm=True, bf16_twin=True)
    skip1 = x2[:n0]

    infl1 = _tc_infl(gpos1, p1T, kpts[1], RADII[1], b1)
    gp1 = _gather_rows(x2bf, pool1f)
    xT, y = _tc_pooldown(gp1, rbs2[0]["wd"], rbs2[0]["bd"], b1, y_rm=True)
    for ri, rb in enumerate(rbs2):
        gy = _gather_rows(y, idx1f)
        final = ri == len(rbs2) - 1
        nxt = None if final else (rbs2[ri + 1]["wd"], rbs2[ri + 1]["bd"])
        res = _tc_resblock(gy, infl1, xT, rb["wf"], rb["bk"], rb["wu"],
                           rb["bu"], rb["wsc"], nxt, None, r1,
                           out_rm=final, bf16_twin=final)
        if final:
            x_l1, x_l1bf = res
        else:
            xT, y = res
    skip2 = x_l1[:n1]

    infl2 = _tc_infl_oh(p2T, idx2T, kpts[2], RADII[2], b2)
    gp2 = _gather_rows(x_l1bf, pool2f)
    xT, yT = _tc_pooldown(gp2, rbs3[0]["wd"], rbs3[0]["bd"], b2, y_rm=False)
    for ri, rb in enumerate(rbs3):
        final = ri == len(rbs3) - 1
        nxt = None if final else (rbs3[ri + 1]["wd"], rbs3[ri + 1]["bd"])
        res = _tc_resblock_oh(yT, idx2T, infl2, xT, rb["wf"], rb["bk"],
                              rb["wu"], rb["bu"], rb["wsc"], nxt, None,
                              r2, out_rm=final)
        if final:
            xT_l2, x_l2 = res
        else:
            xT, yT = res
    skip3 = x_l2[:n2]

    infl3 = _tc_infl_oh(p3T, idx3T, kpts[3], RADII[3], b3)
    xT, yT = _tc_pooldown_oh(xT_l2, pool3T, rbs4[0]["wd"], rbs4[0]["bd"], b3)
    xT, yT = _tc_resblock_oh(yT, idx3T, infl3, xT, rbs4[0]["wf"],
                             rbs4[0]["bk"], rbs4[0]["wu"], rbs4[0]["bu"],
                             rbs4[0]["wsc"],
                             (rbs4[1]["wd"], rbs4[1]["bd"]), None, r3)
    (out,) = _tc_resblock_oh(yT, idx3T, infl3, xT, rbs4[1]["wf"],
                             rbs4[1]["bk"], rbs4[1]["wu"], rbs4[1]["bu"],
                             rbs4[1]["wsc"], None, head, r3)

    return (out[:n3], skip1, skip2, skip3)


# speedup vs baseline: 4.1188x; 1.0118x over previous
import functools

import jax
import jax.numpy as jnp
from jax import lax
from jax.experimental import pallas as pl
from jax.experimental.pallas import tpu as pltpu
from jax.experimental.pallas import tpu_sc as plsc

KP = 15
KN = 16
NW = 32
RADII = (0.0625, 0.125, 0.25, 0.5)
F32 = jnp.float32
BF16 = jnp.bfloat16


def _leaky(x):
    return jnp.where(x >= 0.0, x, 0.1 * x)


def _pad_rows(x, n):
    if x.shape[0] == n:
        return x
    return jnp.pad(x, ((0, n - x.shape[0]),) + ((0, 0),) * (x.ndim - 1))


def _sc_plan(B, D, esize=4):
    bpw = B // NW
    ch = min(128, bpw)
    while bpw % ch:
        ch -= 8
    g = max(1, min(131072 // (ch * D * esize), 8))
    while bpw % (ch * g):
        g -= 1
    sch = ch * g
    return bpw, ch, g, sch, bpw // sch


def _sc_pipeline(table_hbm, idx_hbm, out_hbm, idx_v, buf_a, buf_b,
                 gs_a, gs_b, ws_a, ws_b, base, plan):
    bpw, ch, g, sch, nsch = plan
    pltpu.sync_copy(idx_hbm.at[pl.ds(base, bpw)], idx_v)

    def fire(s, buf, gsem):
        for j in range(g):
            pltpu.async_copy(
                table_hbm.at[idx_v.at[pl.ds(s * sch + j * ch, ch)]],
                buf.at[pl.ds(j * ch, ch)], gsem)

    def retire(s, buf, gsem, wsem):
        pltpu.make_async_copy(table_hbm.at[pl.ds(0, sch)], buf, gsem).wait()
        pltpu.async_copy(buf, out_hbm.at[pl.ds(base + s * sch, sch)],
                         wsem).wait()

    fire(0, buf_a, gs_a)
    if nsch == 1:
        retire(0, buf_a, gs_a, ws_a)
    else:
        @pl.loop(0, nsch // 2)
        def _(h):
            s0 = 2 * h
            fire(s0 + 1, buf_b, gs_b)
            retire(s0, buf_a, gs_a, ws_a)

            @pl.when(s0 + 2 < nsch)
            def _():
                fire(s0 + 2, buf_a, gs_a)
            retire(s0 + 1, buf_b, gs_b, ws_b)
        if nsch % 2:
            retire(nsch - 1, buf_a, gs_a, ws_a)


def _sc_gather_multi(pairs):
    plans = [_sc_plan(idx.shape[0], t.shape[1], t.dtype.itemsize)
             for t, idx in pairs]
    mesh = plsc.VectorSubcoreMesh(core_axis_name="c", subcore_axis_name="s")
    scratch = []
    for (t, idx), (bpw, ch, g, sch, nsch) in zip(pairs, plans):
        scratch += [pltpu.VMEM((bpw,), jnp.int32),
                    pltpu.VMEM((sch, t.shape[1]), t.dtype),
                    pltpu.VMEM((sch, t.shape[1]), t.dtype)]
    scratch += [pltpu.SemaphoreType.DMA] * 4

    @functools.partial(
        pl.kernel, mesh=mesh,
        out_type=[jax.ShapeDtypeStruct((idx.shape[0], t.shape[1]), t.dtype)
                  for t, idx in pairs],
        compiler_params=pltpu.CompilerParams(use_tc_tiling_on_sc=False),
        scratch_types=scratch)
    def gk(*refs):
        n = len(pairs)
        tables = refs[:2 * n:2]
        idxs = refs[1:2 * n:2]
        outs = refs[2 * n:3 * n]
        rest = refs[3 * n:]
        sems = rest[-4:]
        wid = lax.axis_index("s") * 2 + lax.axis_index("c")
        for i, plan in enumerate(plans):
            _sc_pipeline(tables[i], idxs[i], outs[i],
                         rest[3 * i], rest[3 * i + 1], rest[3 * i + 2],
                         sems[0], sems[1], sems[2], sems[3],
                         wid * plan[0], plan)

    flat = []
    for t, idx in pairs:
        flat += [t, idx]
    res = gk(*flat)
    return res if isinstance(res, (list, tuple)) else [res]


def _sc_gather(table, idx_flat):
    _, D = table.shape
    B = idx_flat.shape[0]
    dt = table.dtype
    bpw, ch, g, sch, nsch = _sc_plan(B, D, table.dtype.itemsize)
    mesh = plsc.VectorSubcoreMesh(core_axis_name="c", subcore_axis_name="s")

    @functools.partial(
        pl.kernel, mesh=mesh,
        out_type=jax.ShapeDtypeStruct((B, D), dt),
        compiler_params=pltpu.CompilerParams(use_tc_tiling_on_sc=False),
        scratch_types=[pltpu.VMEM((bpw,), jnp.int32),
                       pltpu.VMEM((sch, D), dt),
                       pltpu.VMEM((sch, D), dt),
                       pltpu.SemaphoreType.DMA,
                       pltpu.SemaphoreType.DMA,
                       pltpu.SemaphoreType.DMA,
                       pltpu.SemaphoreType.DMA])
    def gk(table_hbm, idx_hbm, out_hbm, idx_v, buf_a, buf_b, gs_a, gs_b,
           ws_a, ws_b):
        wid = lax.axis_index("s") * 2 + lax.axis_index("c")
        base = wid * bpw
        pltpu.sync_copy(idx_hbm.at[pl.ds(base, bpw)], idx_v)

        def fire(s, buf, gsem):
            for j in range(g):
                pltpu.async_copy(
                    table_hbm.at[idx_v.at[pl.ds(s * sch + j * ch, ch)]],
                    buf.at[pl.ds(j * ch, ch)], gsem)

        def retire(s, buf, gsem, wsem):
            pltpu.make_async_copy(table_hbm.at[pl.ds(0, sch)], buf, gsem).wait()
            pltpu.async_copy(buf, out_hbm.at[pl.ds(base + s * sch, sch)],
                             wsem).wait()

        fire(0, buf_a, gs_a)
        if nsch == 1:
            retire(0, buf_a, gs_a, ws_a)
        else:
            @pl.loop(0, nsch // 2)
            def _(h):
                s0 = 2 * h
                fire(s0 + 1, buf_b, gs_b)
                retire(s0, buf_a, gs_a, ws_a)

                @pl.when(s0 + 2 < nsch)
                def _():
                    fire(s0 + 2, buf_a, gs_a)
                retire(s0 + 1, buf_b, gs_b, ws_b)
            if nsch % 2:
                retire(nsch - 1, buf_a, gs_a, ws_a)

    return gk(table, idx_flat)


def _gather_rows(table, idx_flat):
    out = _sc_gather(table, idx_flat)
    return out.reshape(idx_flat.shape[0] // KN, KN * table.shape[1])


def _dot(a, b):
    return jax.lax.dot_general(a, b, (((1,), (0,)), ((), ())),
                               preferred_element_type=F32)


def _kp_cols(kpts):
    return tuple(kpts[:, i].reshape(KP, 1) for i in range(3))


def _infl_rows_t(gt, pxt, pyt, pzt, kxc, kyc, kzc, inv_r, stride=16):
    rows = []
    for k in range(KN):
        o = stride * k
        rx = gt[o:o + 1, :] - pxt
        ry = gt[o + 1:o + 2, :] - pyt
        rz = gt[o + 2:o + 3, :] - pzt
        dx = rx - kxc
        dy = ry - kyc
        dz = rz - kzc
        d = jnp.sqrt(dx * dx + dy * dy + dz * dz + 1e-12)
        rows.append(jnp.maximum(0.0, 1.0 - d * inv_r))
    return rows


def _onehot(idxk, v):
    iota = lax.broadcasted_iota(jnp.int32, (v, 1), 0)
    return jnp.where(idxk == iota, 1.0, 0.0).astype(F32)


def _wspec(shape):
    return pl.BlockSpec(shape, lambda i: tuple(0 for _ in shape))


def _cspec(shape):
    return pl.BlockSpec(shape, lambda i: (0, i))


def _rspec(shape):
    return pl.BlockSpec(shape, lambda i: (i, 0))


def _tc_level0(g0, tT, kpts, w0, b0, wd, bd, blk):
    n = tT.shape[1]
    inv_r = 1.0 / RADII[0]
    c0, m = w0.shape[1], wd.shape[1]
    kxc, kyc, kzc = _kp_cols(kpts)

    def body(g_ref, t_ref, kx_ref, ky_ref, kz_ref, w0_ref, b0_ref,
             wd_ref, bd_ref, infl_ref, x0_ref, y1_ref):
        gt = jnp.transpose(g_ref[...])
        t = t_ref[...]
        rows = _infl_rows_t(gt, t[0:1, :], t[1:2, :], t[2:3, :],
                            kx_ref[...], ky_ref[...], kz_ref[...], inv_r)
        agg = None
        for k in range(KN):
            a = rows[k] * gt[16 * k + 3:16 * k + 4, :]
            agg = a if agg is None else agg + a
        infl_ref[...] = jnp.concatenate(rows, axis=0)
        x0 = _leaky(_dot(w0_ref[...], agg) + b0_ref[...])
        x0_ref[...] = x0
        y1_ref[...] = jnp.transpose(
            _leaky(_dot(wd_ref[...], x0) + bd_ref[...])).astype(BF16)

    return pl.pallas_call(
        body,
        grid=(n // blk,),
        in_specs=[_rspec((blk, KN * 16)), _cspec((16, blk)),
                  _wspec((KP, 1)), _wspec((KP, 1)), _wspec((KP, 1)),
                  _wspec((c0, KP)), _wspec((c0, 1)),
                  _wspec((m, c0)), _wspec((m, 1))],
        out_specs=[_cspec((KN * KP, blk)), _cspec((c0, blk)),
                   _rspec((blk, m))],
        out_shape=[jax.ShapeDtypeStruct((KN * KP, n), F32),
                   jax.ShapeDtypeStruct((c0, n), F32),
                   jax.ShapeDtypeStruct((n, m), BF16)],
    )(g0, tT, kxc, kyc, kzc, w0.T, b0.T, wd.T, bd.T)


def _tc_infl(gpos, pT, kpts, radius, blk):
    n = pT.shape[1]
    inv_r = 1.0 / radius
    kxc, kyc, kzc = _kp_cols(kpts)

    def body(g_ref, t_ref, kx_ref, ky_ref, kz_ref, infl_ref):
        gt = jnp.transpose(g_ref[...])
        t = t_ref[...]
        rows = _infl_rows_t(gt, t[0:1, :], t[1:2, :], t[2:3, :],
                            kx_ref[...], ky_ref[...], kz_ref[...], inv_r)
        infl_ref[...] = jnp.concatenate(rows, axis=0)

    return pl.pallas_call(
        body,
        grid=(n // blk,),
        in_specs=[_rspec((blk, KN * 16)), _cspec((16, blk)),
                  _wspec((KP, 1)), _wspec((KP, 1)), _wspec((KP, 1))],
        out_specs=_cspec((KN * KP, blk)),
        out_shape=jax.ShapeDtypeStruct((KN * KP, n), F32),
    )(gpos, pT, kxc, kyc, kzc)


def _tc_infl_oh(pT, idxT, kpts, radius, blk):
    v = pT.shape[1]
    n = idxT.shape[1]
    inv_r = 1.0 / radius
    kxc, kyc, kzc = _kp_cols(kpts)

    def body(p_ref, pc_ref, i_ref, kx_ref, ky_ref, kz_ref, infl_ref):
        pt = p_ref[...][0:8, :]
        pc = pc_ref[...]
        idx = i_ref[...]
        pxt, pyt, pzt = pc[0:1, :], pc[1:2, :], pc[2:3, :]
        kxcv, kycv, kzcv = kx_ref[...], ky_ref[...], kz_ref[...]
        rows = []
        for k in range(KN):
            gk = _dot(pt, _onehot(idx[k:k + 1, :], v))
            rx = gk[0:1, :] - pxt
            ry = gk[1:2, :] - pyt
            rz = gk[2:3, :] - pzt
            dx = rx - kxcv
            dy = ry - kycv
            dz = rz - kzcv
            d = jnp.sqrt(dx * dx + dy * dy + dz * dz + 1e-12)
            rows.append(jnp.maximum(0.0, 1.0 - d * inv_r))
        infl_ref[...] = jnp.concatenate(rows, axis=0)

    return pl.pallas_call(
        body,
        grid=(n // blk,),
        in_specs=[_wspec((16, v)), _cspec((16, blk)), _cspec((KN, blk)),
                  _wspec((KP, 1)), _wspec((KP, 1)), _wspec((KP, 1))],
        out_specs=_cspec((KN * KP, blk)),
        out_shape=jax.ShapeDtypeStruct((KN * KP, n), F32),
    )(pT, pT, idxT, kxc, kyc, kzc)


def _tc_pooldown(gp, wd, bd, blk, y_rm):
    n = gp.shape[0]
    cp, m = wd.shape

    def body(g_ref, wd_ref, bd_ref, xp_ref, y_ref):
        g = g_ref[...].astype(F32)
        xp = g[:, :cp]
        for k in range(1, KN):
            xp = jnp.maximum(xp, g[:, k * cp:(k + 1) * cp])
        xpt = jnp.transpose(xp)
        xp_ref[...] = xpt
        yt = _leaky(_dot(wd_ref[...], xpt) + bd_ref[...])
        y_ref[...] = jnp.transpose(yt).astype(BF16) if y_rm else yt

    y_spec = _rspec((blk, m)) if y_rm else _cspec((m, blk))
    y_shape, y_dt = ((n, m), BF16) if y_rm else ((m, n), F32)
    return pl.pallas_call(
        body,
        grid=(n // blk,),
        in_specs=[_rspec((blk, KN * cp)), _wspec((m, cp)), _wspec((m, 1))],
        out_specs=[_cspec((cp, blk)), y_spec],
        out_shape=[jax.ShapeDtypeStruct((cp, n), F32),
                   jax.ShapeDtypeStruct(y_shape, y_dt)],
    )(gp, wd.T, bd.T)


def _tc_pooldown_oh(xT, poolT, wd, bd, blk):
    cp, v = xT.shape
    n = poolT.shape[1]
    m = wd.shape[1]

    def body(x_ref, i_ref, wd_ref, bd_ref, xp_ref, y_ref):
        xt = x_ref[...]
        idx = i_ref[...]
        xp = None
        for k in range(KN):
            gk = _dot(xt, _onehot(idx[k:k + 1, :], v))
            xp = gk if xp is None else jnp.maximum(xp, gk)
        xp_ref[...] = xp
        y_ref[...] = _leaky(_dot(wd_ref[...], xp) + bd_ref[...])

    return pl.pallas_call(
        body,
        grid=(n // blk,),
        in_specs=[_wspec((cp, v)), _cspec((KN, blk)),
                  _wspec((m, cp)), _wspec((m, 1))],
        out_specs=[_cspec((cp, blk)), _cspec((m, blk))],
        out_shape=[jax.ShapeDtypeStruct((cp, n), F32),
                   jax.ShapeDtypeStruct((m, n), F32)],
    )(xT, poolT, wd.T, bd.T)


def _agg_tail(gk_of_k, iv, wf_ref, bk_ref, wu_ref, bu_ref, wsc_ref, xt):
    accs = [None] * KP
    for k in range(KN):
        gk = gk_of_k(k)
        for p in range(KP):
            w = iv[k * KP + p:k * KP + p + 1, :]
            t = w * gk
            accs[p] = t if accs[p] is None else accs[p] + t
    s = jnp.concatenate(accs, axis=0)
    z = _leaky(_dot(wf_ref[...], s) + bk_ref[...])
    o = _dot(wu_ref[...], z) + bu_ref[...]
    sc = _dot(wsc_ref[...], xt) if wsc_ref is not None else xt
    return _leaky(o + sc)


def _tc_resblock(gy, inflT, xT_in, wf, bk, wu, bu, wsc, dn, head, blk,
                 out_rm=False, bf16_twin=False, infl_off=0, x_off=0):
    n = gy.shape[0]
    m = wf.shape[1]
    ci = xT_in.shape[0]
    co = wu.shape[1]
    has_sc = wsc is not None
    wdn, bdn = dn if dn is not None else (None, None)
    wh, bh = head if head is not None else (None, None)

    def body(*refs):
        it = iter(refs)
        g_ref, infl_ref, x_ref, wf_ref, bk_ref, wu_ref, bu_ref = (
            next(it) for _ in range(7))
        wsc_ref = next(it) if has_sc else None
        wdn_ref, bdn_ref = (next(it), next(it)) if dn is not None else (0, 0)
        wh_ref, bh_ref = (next(it), next(it)) if head is not None else (0, 0)
        outs = list(it)

        gt = jnp.transpose(g_ref[...].astype(F32))
        xo = _agg_tail(lambda k: gt[k * m:(k + 1) * m, :], infl_ref[...],
                       wf_ref, bk_ref, wu_ref, bu_ref, wsc_ref, x_ref[...])
        if head is not None:
            outs[0][...] = jnp.transpose(_dot(wh_ref[...], xo) + bh_ref[...])
            return
        outs[0][...] = jnp.transpose(xo) if out_rm else xo
        if bf16_twin:
            outs[1][...] = jnp.transpose(xo).astype(BF16)
        if dn is not None:
            outs[1 + bf16_twin][...] = jnp.transpose(
                _leaky(_dot(wdn_ref[...], xo) + bdn_ref[...])).astype(BF16)

    in_specs = [_rspec((blk, KN * m)),
                pl.BlockSpec((KN * KP, blk), lambda i: (0, i + infl_off)),
                pl.BlockSpec((ci, blk), lambda i: (0, i + x_off)),
                _wspec((m, KP * m)), _wspec((m, 1)),
                _wspec((co, m)), _wspec((co, 1))]
    args = [gy, inflT, xT_in, wf.T, bk.T, wu.T, bu.T]
    if has_sc:
        in_specs.append(_wspec((co, ci)))
        args.append(wsc.T)
    if dn is not None:
        mn = wdn.shape[1]
        in_specs += [_wspec((mn, co)), _wspec((mn, 1))]
        args += [wdn.T, bdn.T]
    if head is not None:
        chh = wh.shape[1]
        in_specs += [_wspec((chh, co)), _wspec((chh, 1))]
        args += [wh.T, bh.T]

    if head is not None:
        out_specs = [_rspec((blk, wh.shape[1]))]
        out_shape = [jax.ShapeDtypeStruct((n, wh.shape[1]), F32)]
    else:
        out_specs = [_rspec((blk, co)) if out_rm else _cspec((co, blk))]
        out_shape = [jax.ShapeDtypeStruct((n, co) if out_rm else (co, n), F32)]
        if bf16_twin:
            out_specs.append(_rspec((blk, co)))
            out_shape.append(jax.ShapeDtypeStruct((n, co), BF16))
        if dn is not None:
            mn = wdn.shape[1]
            out_specs.append(_rspec((blk, mn)))
            out_shape.append(jax.ShapeDtypeStruct((n, mn), BF16))

    return pl.pallas_call(body, grid=(n // blk,), in_specs=in_specs,
                          out_specs=out_specs, out_shape=out_shape)(*args)


def _tc_resblock_oh(yT, idxT, inflT, xT_in, wf, bk, wu, bu, wsc, dn, head,
                    blk, out_rm=False):
    m, v = yT.shape
    n = idxT.shape[1]
    ci = xT_in.shape[0]
    co = wu.shape[1]
    has_sc = wsc is not None
    wdn, bdn = dn if dn is not None else (None, None)
    wh, bh = head if head is not None else (None, None)

    def body(*refs):
        it = iter(refs)
        y_ref, i_ref, infl_ref, x_ref, wf_ref, bk_ref, wu_ref, bu_ref = (
            next(it) for _ in range(8))
        wsc_ref = next(it) if has_sc else None
        wdn_ref, bdn_ref = (next(it), next(it)) if dn is not None else (0, 0)
        wh_ref, bh_ref = (next(it), next(it)) if head is not None else (0, 0)
        outs = list(it)

        yt = y_ref[...]
        idx = i_ref[...]
        xo = _agg_tail(lambda k: _dot(yt, _onehot(idx[k:k + 1, :], v)),
                       infl_ref[...], wf_ref, bk_ref, wu_ref, bu_ref,
                       wsc_ref, x_ref[...])
        if head is not None:
            outs[0][...] = jnp.transpose(_dot(wh_ref[...], xo) + bh_ref[...])
            return
        outs[0][...] = xo
        i = 1
        if out_rm:
            outs[i][...] = jnp.transpose(xo)
            i += 1
        if dn is not None:
            outs[i][...] = _leaky(_dot(wdn_ref[...], xo) + bdn_ref[...])

    in_specs = [_wspec((m, v)), _cspec((KN, blk)), _cspec((KN * KP, blk)),
                _cspec((ci, blk)),
                _wspec((m, KP * m)), _wspec((m, 1)),
                _wspec((co, m)), _wspec((co, 1))]
    args = [yT, idxT, inflT, xT_in, wf.T, bk.T, wu.T, bu.T]
    if has_sc:
        in_specs.append(_wspec((co, ci)))
        args.append(wsc.T)
    if dn is not None:
        mn = wdn.shape[1]
        in_specs += [_wspec((mn, co)), _wspec((mn, 1))]
        args += [wdn.T, bdn.T]
    if head is not None:
        chh = wh.shape[1]
        in_specs += [_wspec((chh, co)), _wspec((chh, 1))]
        args += [wh.T, bh.T]

    if head is not None:
        out_specs = [_rspec((blk, wh.shape[1]))]
        out_shape = [jax.ShapeDtypeStruct((n, wh.shape[1]), F32)]
    else:
        out_specs = [_cspec((co, blk))]
        out_shape = [jax.ShapeDtypeStruct((co, n), F32)]
        if out_rm:
            out_specs.append(_rspec((blk, co)))
            out_shape.append(jax.ShapeDtypeStruct((n, co), F32))
        if dn is not None:
            mn = wdn.shape[1]
            out_specs.append(_cspec((mn, blk)))
            out_shape.append(jax.ShapeDtypeStruct((mn, n), F32))

    return pl.pallas_call(body, grid=(n // blk,), in_specs=in_specs,
                          out_specs=out_specs, out_shape=out_shape)(*args)


def _fold_lin(w, bn):
    return w * bn["g"][None, :], bn["b"].reshape(1, -1)


def _fold_kp(kpw, bn):
    pm, m = kpw.shape[0] * kpw.shape[1], kpw.shape[2]
    return kpw.reshape(pm, m) * bn["g"][None, :], bn["b"].reshape(1, -1)


def _rb_weights(p):
    wd, bd = _fold_lin(p["down"], p["bnd"])
    wf, bk = _fold_kp(p["kp"], p["bnk"])
    wu, bu = _fold_lin(p["up"], p["bnu"])
    return dict(wd=wd, bd=bd, wf=wf, bk=bk, wu=wu, bu=bu, wsc=p.get("sc"))


def kernel(points, features, idx0, idx1, idx2, idx3, pool1, pool2, pool3,
           pos1, pos2, pos3, params):
    n0, n1, n2, n3 = points.shape[0], pos1.shape[0], pos2.shape[0], pos3.shape[0]
    np0, np1, np2, np3 = 10240, 2560, 640, 160
    b0, b1, b2, b3 = 256, 256, 128, 160
    r0, r1, r2, r3 = 1024, 512, 128, 160

    def padw(x, w=16):
        return jnp.pad(x, ((0, 0), (0, w - x.shape[1])))

    t0 = _pad_rows(padw(jnp.concatenate([points, features], axis=1)), np0)
    p1 = _pad_rows(padw(pos1), np1)
    p2 = _pad_rows(padw(pos2), np2)
    p3 = _pad_rows(padw(pos3), np3)
    t0T, p1T, p2T, p3T = t0.T, p1.T, p2.T, p3.T
    idx0f = _pad_rows(idx0, np0).reshape(-1)
    idx1f = _pad_rows(idx1, np1).reshape(-1)
    idx2T = _pad_rows(idx2, np2).T
    idx3T = idx3.T
    pool1f = _pad_rows(pool1, np1).reshape(-1)
    pool2f = _pad_rows(pool2, np2).reshape(-1)
    pool3T = pool3.T

    kpts = params["kpts"]
    w0, bb0 = _fold_kp(params["b1_kp"].reshape(KP, 1, 64), params["b1_bn"])
    rb_b1 = _rb_weights(params["b1_rb"])
    rb_a1 = _rb_weights(params["b1_ra"])
    rbs2 = [_rb_weights(p) for p in params["b2"]]
    rbs3 = [_rb_weights(p) for p in params["b3"]]
    rbs4 = [_rb_weights(p) for p in params["b4"]]
    head = (params["head_w"], params["head_b"].reshape(1, -1))

    g0r, gpos1r = _sc_gather_multi([(t0, idx0f), (p1, idx1f)])
    g0 = g0r.reshape(np0, KN * 16)
    gpos1 = gpos1r.reshape(np1, KN * 16)
    infl0, x0T, y = _tc_level0(g0, t0T, kpts[0], w0, bb0,
                               rb_b1["wd"], rb_b1["bd"], b0)
    h0 = np0 // 2
    hb0 = h0 // r0
    idx0a, idx0b = idx0f[:h0 * KN], idx0f[h0 * KN:]
    gya = _gather_rows(y, idx0a)
    gyb = _gather_rows(y, idx0b)
    x1ta, ya = _tc_resblock(gya, infl0, x0T, rb_b1["wf"], rb_b1["bk"],
                            rb_b1["wu"], rb_b1["bu"], rb_b1["wsc"],
                            (rb_a1["wd"], rb_a1["bd"]), None, r0)
    x1tb, yb = _tc_resblock(gyb, infl0, x0T, rb_b1["wf"], rb_b1["bk"],
                            rb_b1["wu"], rb_b1["bu"], rb_b1["wsc"],
                            (rb_a1["wd"], rb_a1["bd"]), None, r0,
                            infl_off=hb0, x_off=hb0)
    y = jnp.concatenate([ya, yb], axis=0)
    gya = _gather_rows(y, idx0a)
    gyb = _gather_rows(y, idx0b)
    x2a, x2bfa = _tc_resblock(gya, infl0, x1ta, rb_a1["wf"], rb_a1["bk"],
                              rb_a1["wu"], rb_a1["bu"], rb_a1["wsc"],
                              None, None, r0, out_rm=True, bf16_twin=True)
    x2b, x2bfb = _tc_resblock(gyb, infl0, x1tb, rb_a1["wf"], rb_a1["bk"],
                              rb_a1["wu"], rb_a1["bu"], rb_a1["wsc"],
                              None, None, r0, out_rm=True, bf16_twin=True,
                              infl_off=hb0)
    x2 = jnp.concatenate([x2a, x2b], axis=0)
    x2bf = jnp.concatenate([x2bfa, x2bfb], axis=0)
    skip1 = x2[:n0]

    infl1 = _tc_infl(gpos1, p1T, kpts[1], RADII[1], b1)
    gp1 = _gather_rows(x2bf, pool1f)
    xT, y = _tc_pooldown(gp1, rbs2[0]["wd"], rbs2[0]["bd"], b1, y_rm=True)
    h1 = 1536
    hb1 = h1 // r1
    idx1a, idx1b = idx1f[:h1 * KN], idx1f[h1 * KN:]
    xa = xb = xT
    xoff_a, xoff_b = 0, hb1
    for ri, rb in enumerate(rbs2):
        gya = _gather_rows(y, idx1a)
        gyb = _gather_rows(y, idx1b)
        final = ri == len(rbs2) - 1
        nxt = None if final else (rbs2[ri + 1]["wd"], rbs2[ri + 1]["bd"])
        resa = _tc_resblock(gya, infl1, xa, rb["wf"], rb["bk"], rb["wu"],
                            rb["bu"], rb["wsc"], nxt, None, r1,
                            out_rm=final, bf16_twin=final, x_off=xoff_a)
        resb = _tc_resblock(gyb, infl1, xb, rb["wf"], rb["bk"], rb["wu"],
                            rb["bu"], rb["wsc"], nxt, None, r1,
                            out_rm=final, bf16_twin=final,
                            infl_off=hb1, x_off=xoff_b)
        if final:
            x_l1 = jnp.concatenate([resa[0], resb[0]], axis=0)
            x_l1bf = jnp.concatenate([resa[1], resb[1]], axis=0)
        else:
            xa, ya = resa
            xb, yb = resb
            xoff_a = xoff_b = 0
            y = jnp.concatenate([ya, yb], axis=0)
    skip2 = x_l1[:n1]

    infl2 = _tc_infl_oh(p2T, idx2T, kpts[2], RADII[2], b2)
    gp2 = _gather_rows(x_l1bf, pool2f)
    xT, yT = _tc_pooldown(gp2, rbs3[0]["wd"], rbs3[0]["bd"], b2, y_rm=False)
    for ri, rb in enumerate(rbs3):
        final = ri == len(rbs3) - 1
        nxt = None if final else (rbs3[ri + 1]["wd"], rbs3[ri + 1]["bd"])
        res = _tc_resblock_oh(yT, idx2T, infl2, xT, rb["wf"], rb["bk"],
                              rb["wu"], rb["bu"], rb["wsc"], nxt, None,
                              r2, out_rm=final)
        if final:
            xT_l2, x_l2 = res
        else:
            xT, yT = res
    skip3 = x_l2[:n2]

    infl3 = _tc_infl_oh(p3T, idx3T, kpts[3], RADII[3], b3)
    xT, yT = _tc_pooldown_oh(xT_l2, pool3T, rbs4[0]["wd"], rbs4[0]["bd"], b3)
    xT, yT = _tc_resblock_oh(yT, idx3T, infl3, xT, rbs4[0]["wf"],
                             rbs4[0]["bk"], rbs4[0]["wu"], rbs4[0]["bu"],
                             rbs4[0]["wsc"],
                             (rbs4[1]["wd"], rbs4[1]["bd"]), None, r3)
    (out,) = _tc_resblock_oh(yT, idx3T, infl3, xT, rbs4[1]["wf"],
                             rbs4[1]["bk"], rbs4[1]["wu"], rbs4[1]["bu"],
                             rbs4[1]["wsc"], None, head, r3)

    return (out[:n3], skip1, skip2, skip3)
